# Initial kernel scaffold; baseline (speedup 1.0000x reference)
import jax
import jax.numpy as jnp
from jax.experimental import pallas as pl


def kernel(Corpus_, batch_inputs, edge_list, edge_type, train_indices_nhop, entity_embeddings, relation_embeddings, W_entities, W_spgat, a_heads, a2_heads, a_out, a2_out):
    raise NotImplementedError("write your pallas kernel here")


# speedup vs baseline: 1.2525x; 1.2525x over previous
import jax
import jax.numpy as jnp
from jax import lax
from jax.experimental import pallas as pl
from jax.experimental.pallas import tpu as pltpu
from jax.experimental.pallas import tpu_sc as plsc

N = 10000
E = 160000
NHOP = 40000
RN = 500
ALPHA = 0.2

NC = 2
NS = 16
NW = NC * NS

NPAD = 10240
RPAD = 512
EPAD = 163840
HPAD = 40960
CH = 64
D = 128
ROWS_PER_TILE = NPAD // NS
ET_N, EC_N = EPAD // NW, EPAD // NC
ET_H, EC_H = HPAD // NW, HPAD // NC
BT = 4096 // NW


def _leakyexp(z):
    return jnp.exp(-jnp.where(z >= 0, z, ALPHA * z))


def _zero_rows(buf, width, nrows):
    def body(i, c):
        for d in range(width // 16):
            buf[i, pl.ds(d * 16, 16)] = jnp.zeros((16,), jnp.float32)
        return c
    lax.fori_loop(0, nrows, body, 0)


def _init_accum(accum_n, accum_w, out_rows, w_rows, sid):
    _zero_rows(out_rows, D, CH)
    _zero_rows(w_rows, 16, CH)
    for j in range(ROWS_PER_TILE // CH):
        st = sid * ROWS_PER_TILE + j * CH
        pltpu.sync_copy(out_rows, accum_n.at[pl.ds(st, CH)])
        pltpu.sync_copy(w_rows, accum_w.at[pl.ds(st, CH)])


def _readout(accum_n, accum_w, out_rows, w_rows, out_n, out_w, cid, sid):
    for j in range(ROWS_PER_TILE // CH):
        st = sid * ROWS_PER_TILE + j * CH
        pltpu.sync_copy(accum_n.at[pl.ds(st, CH)], out_rows)
        pltpu.sync_copy(out_rows, out_n.at[pl.ds(cid * NPAD + st, CH)])
        pltpu.sync_copy(accum_w.at[pl.ds(st, CH)], w_rows)
        pltpu.sync_copy(w_rows, out_w.at[pl.ds(cid * NPAD + st, CH)])


def _att_body(nheads,
              dst_n, src_n, rt_n, dst_h, src_h, bidx_h,
              psrc_hbm, prel_hbm, preln_hbm,
              tdst_hbm, tsrc_hbm, trel_hbm, tsrn_hbm,
              out_n, out_w,
              accum_n, accum_w, dst_v, src_v, ra_v,
              rows_s, rows_a, out_rows, w_rows,
              dscal, sscal, rscal,
              sem_a, sem_b, sem_c, sem_d, sem_e):
    cid = lax.axis_index("c")
    sid = lax.axis_index("s")
    lane = lax.iota(jnp.int32, 16)
    zeros16 = jnp.zeros((16,), jnp.int32)
    _init_accum(accum_n, accum_w, out_rows, w_rows, sid)
    plsc.subcore_barrier()

    if bidx_h is not None:
        def mrow(i, c):
            w_rows[i, pl.ds(0, 16)] = jnp.where(lane == 1, 1.0, 0.0)
            return c
        lax.fori_loop(0, CH, mrow, 0)
        wid = cid * NS + sid
        for j in range(BT // CH):
            pltpu.sync_copy(bidx_h.at[pl.ds(wid * BT + j * CH, CH)], dst_v)
            pltpu.sync_copy(w_rows, accum_w.at[dst_v], add=True)

    def scale_phase():
        def eb(e, c):
            sl = pl.ds(0, 16)
            z16 = dscal[e, sl] + sscal[e, sl] + rscal[e, sl]
            w16 = _leakyexp(z16)
            w_rows[e, sl] = jnp.where(lane < nheads, w16, 0.0)
            esplat = jnp.full((16,), e, jnp.int32)
            w0 = plsc.load_gather(w_rows, [esplat, zeros16])
            if nheads == 2:
                w1 = plsc.load_gather(w_rows, [esplat, zeros16 + 1])
            for d in range(8):
                sld = pl.ds(d * 16, 16)
                v = rows_s[e, sld] + rows_a[e, sld]
                if nheads == 2:
                    out_rows[e, sld] = v * (w0 if d < 4 else w1)
                else:
                    out_rows[e, sld] = v * w0
            return c
        lax.fori_loop(0, CH, eb, 0)

    def nchunk(i, c):
        base = cid * EC_N + sid * ET_N + i * CH
        pltpu.sync_copy(dst_n.at[pl.ds(base, CH)], dst_v)
        pltpu.sync_copy(src_n.at[pl.ds(base, CH)], src_v)
        pltpu.sync_copy(rt_n.at[pl.ds(base, CH)], ra_v)
        cp1 = pltpu.async_copy(psrc_hbm.at[src_v], rows_s, sem_a)
        cp2 = pltpu.async_copy(prel_hbm.at[ra_v], rows_a, sem_b)
        cp3 = pltpu.async_copy(tdst_hbm.at[dst_v], dscal, sem_c)
        cp4 = pltpu.async_copy(tsrc_hbm.at[src_v], sscal, sem_d)
        cp5 = pltpu.async_copy(trel_hbm.at[ra_v], rscal, sem_e)
        cp1.wait()
        cp2.wait()
        cp3.wait()
        cp4.wait()
        cp5.wait()
        scale_phase()
        pltpu.sync_copy(out_rows, accum_n.at[dst_v], add=True)
        pltpu.sync_copy(w_rows, accum_w.at[dst_v], add=True)
        return c
    lax.fori_loop(0, ET_N // CH, nchunk, 0)

    def hchunk(i, c):
        gbase = cid * EC_H + sid * ET_H + i * CH
        pltpu.sync_copy(dst_h.at[pl.ds(gbase, CH)], dst_v)
        pltpu.sync_copy(src_h.at[pl.ds(gbase, CH)], src_v)
        cp1 = pltpu.async_copy(psrc_hbm.at[src_v], rows_s, sem_a)
        cp2 = pltpu.async_copy(preln_hbm.at[pl.ds(gbase, CH)], rows_a, sem_b)
        cp3 = pltpu.async_copy(tdst_hbm.at[dst_v], dscal, sem_c)
        cp4 = pltpu.async_copy(tsrc_hbm.at[src_v], sscal, sem_d)
        cp5 = pltpu.async_copy(tsrn_hbm.at[pl.ds(gbase, CH)], rscal, sem_e)
        cp1.wait()
        cp2.wait()
        cp3.wait()
        cp4.wait()
        cp5.wait()
        scale_phase()
        pltpu.sync_copy(out_rows, accum_n.at[dst_v], add=True)
        pltpu.sync_copy(w_rows, accum_w.at[dst_v], add=True)
        return c
    lax.fori_loop(0, ET_H // CH, hchunk, 0)

    plsc.subcore_barrier()
    _readout(accum_n, accum_w, out_rows, w_rows, out_n, out_w, cid, sid)


def _att1_body(dst_n, src_n, rt_n, dst_h, src_h,
               psrc_hbm, prel_hbm, preln_hbm,
               tdst_hbm, tsrc_hbm, trel_hbm, tsrn_hbm,
               out_n, out_w, *rest):
    _att_body(2, dst_n, src_n, rt_n, dst_h, src_h, None,
              psrc_hbm, prel_hbm, preln_hbm,
              tdst_hbm, tsrc_hbm, trel_hbm, tsrn_hbm,
              out_n, out_w, *rest)


def _att2_body(dst_n, src_n, rt_n, dst_h, src_h, bidx_h,
               psrc_hbm, prel_hbm, preln_hbm,
               tdst_hbm, tsrc_hbm, trel_hbm, tsrn_hbm,
               out_n, out_w, *rest):
    _att_body(1, dst_n, src_n, rt_n, dst_h, src_h, bidx_h,
              psrc_hbm, prel_hbm, preln_hbm,
              tdst_hbm, tsrc_hbm, trel_hbm, tsrn_hbm,
              out_n, out_w, *rest)


def _mesh():
    return plsc.VectorSubcoreMesh(core_axis_name="c", subcore_axis_name="s")


_OUT_TYPE = (jax.ShapeDtypeStruct((NC * NPAD, D), jnp.float32),
             jax.ShapeDtypeStruct((NC * NPAD, 16), jnp.float32))

_SCRATCH = [
    pltpu.VMEM_SHARED((NPAD, D), jnp.float32),
    pltpu.VMEM_SHARED((NPAD, 16), jnp.float32),
    pltpu.VMEM((CH,), jnp.int32),
    pltpu.VMEM((CH,), jnp.int32),
    pltpu.VMEM((CH,), jnp.int32),
    pltpu.VMEM((CH, D), jnp.float32),
    pltpu.VMEM((CH, D), jnp.float32),
    pltpu.VMEM((CH, D), jnp.float32),
    pltpu.VMEM((CH, 16), jnp.float32),
    pltpu.VMEM((CH, 16), jnp.float32),
    pltpu.VMEM((CH, 16), jnp.float32),
    pltpu.VMEM((CH, 16), jnp.float32),
    pltpu.SemaphoreType.DMA,
    pltpu.SemaphoreType.DMA,
    pltpu.SemaphoreType.DMA,
    pltpu.SemaphoreType.DMA,
    pltpu.SemaphoreType.DMA,
]

_att1 = pl.kernel(
    _att1_body, mesh=_mesh(),
    compiler_params=pltpu.CompilerParams(
        needs_layout_passes=False, use_tc_tiling_on_sc=False),
    out_type=_OUT_TYPE, scratch_types=_SCRATCH)

_att2 = pl.kernel(
    _att2_body, mesh=_mesh(),
    compiler_params=pltpu.CompilerParams(
        needs_layout_passes=False, use_tc_tiling_on_sc=False),
    out_type=_OUT_TYPE, scratch_types=_SCRATCH)


def _normalize_rows(x):
    n = jnp.linalg.norm(x, axis=1, keepdims=True)
    return x / jnp.maximum(n, 1e-12)


def _pad_to(x, n, value=0):
    return jnp.pad(x, [(0, n - x.shape[0])] + [(0, 0)] * (x.ndim - 1),
                   constant_values=value)


def _scal16(*cols):
    rows = cols[0].shape[0]
    out = jnp.zeros((rows, 16), jnp.float32)
    for i, c in enumerate(cols):
        out = out.at[:, i].set(c)
    return out


@jax.jit
def kernel(Corpus_, batch_inputs, edge_list, edge_type, train_indices_nhop,
           entity_embeddings, relation_embeddings, W_entities, W_spgat,
           a_heads, a2_heads, a_out, a2_out):
    f32 = jnp.float32
    ent = _normalize_rows(entity_embeddings)
    rel = _normalize_rows(relation_embeddings)
    rel_aug = jnp.concatenate([rel, jnp.zeros((RPAD - RN, 128), f32)], axis=0)

    i32 = jnp.int32
    dst_n = _pad_to(edge_list[0].astype(i32), EPAD, N)
    src_n = _pad_to(edge_list[1].astype(i32), EPAD, 0)
    rt_n = _pad_to(edge_type.astype(i32), EPAD, RN)
    tin = train_indices_nhop.astype(i32)
    dst_h = _pad_to(tin[:, 3], HPAD, N)
    src_h = _pad_to(tin[:, 0], HPAD, 0)
    ra_h = tin[:, 1]
    rb_h = tin[:, 2]
    bidx = batch_inputs[:, 2].astype(i32)

    p_dst = jnp.concatenate(
        [ent @ a_heads[0, :, :128].T, ent @ a_heads[1, :, :128].T], axis=1)
    p_src = jnp.concatenate(
        [ent @ a_heads[0, :, 128:256].T, ent @ a_heads[1, :, 128:256].T], axis=1)
    p_rel = jnp.concatenate(
        [rel_aug @ a_heads[0, :, 256:].T, rel_aug @ a_heads[1, :, 256:].T], axis=1)
    p_reln = _pad_to(p_rel[ra_h] + p_rel[rb_h], HPAD)
    sd0 = p_dst[:, :64] @ a2_heads[0, 0]
    sd1 = p_dst[:, 64:] @ a2_heads[1, 0]
    ss0 = p_src[:, :64] @ a2_heads[0, 0]
    ss1 = p_src[:, 64:] @ a2_heads[1, 0]
    sr0 = p_rel[:, :64] @ a2_heads[0, 0]
    sr1 = p_rel[:, 64:] @ a2_heads[1, 0]
    tdst = _pad_to(_scal16(sd0, sd1), NPAD)
    tsrc = _pad_to(_scal16(ss0, ss1), NPAD)
    trel = _scal16(sr0, sr1)
    tsrn = _pad_to(_scal16(sr0[ra_h] + sr0[rb_h], sr1[ra_h] + sr1[rb_h]), HPAD)

    acc_n, acc_w = _att1(dst_n, src_n, rt_n, dst_h, src_h,
                         p_src, p_rel, p_reln, tdst, tsrc, trel, tsrn)
    acc_n = acc_n[:NPAD] + acc_n[NPAD:]
    acc_w = acc_w[:NPAD] + acc_w[NPAD:]
    r0 = acc_w[:N, 0:1]
    r1 = acc_w[:N, 1:2]
    h0 = (p_dst[:, :64] * r0 + acc_n[:N, :64]) / jnp.where(r0 == 0.0, 1e-12, r0)
    h1 = (p_dst[:, 64:] * r1 + acc_n[:N, 64:]) / jnp.where(r1 == 0.0, 1e-12, r1)
    x = jnp.concatenate([jax.nn.elu(h0), jax.nn.elu(h1)], axis=1)

    out_relation_1 = rel @ W_spgat
    orel_aug = jnp.concatenate(
        [out_relation_1, jnp.zeros((RPAD - RN, 128), f32)], axis=0)
    q_dst = x @ a_out[:, :128].T
    q_src = x @ a_out[:, 128:256].T
    q_rel = orel_aug @ a_out[:, 256:].T
    q_reln = _pad_to(q_rel[ra_h] + q_rel[rb_h], HPAD)
    s2d = q_dst @ a2_out[0]
    s2s = q_src @ a2_out[0]
    s2r = q_rel @ a2_out[0]
    tdst2 = _pad_to(_scal16(s2d), NPAD)
    tsrc2 = _pad_to(_scal16(s2s), NPAD)
    trel2 = _scal16(s2r)
    tsrn2 = _pad_to(_scal16(s2r[ra_h] + s2r[rb_h]), HPAD)

    acc2_n, acc2_w = _att2(dst_n, src_n, rt_n, dst_h, src_h, bidx,
                           q_src, q_rel, q_reln, tdst2, tsrc2, trel2, tsrn2)
    acc2_n = acc2_n[:NPAD] + acc2_n[NPAD:]
    acc2_w = acc2_w[:NPAD] + acc2_w[NPAD:]
    r2 = acc2_w[:N, 0:1]
    h2 = (q_dst * r2 + acc2_n[:N]) / jnp.where(r2 == 0.0, 1e-12, r2)
    x2 = jax.nn.elu(h2)
    mask = (acc2_w[:N, 1:2] > 0.0).astype(f32)

    out_entity_1 = _normalize_rows(ent @ W_entities + mask * x2)
    return (out_entity_1, out_relation_1)


# speedup vs baseline: 2.0378x; 1.6270x over previous
import jax
import jax.numpy as jnp
from jax import lax
from jax.experimental import pallas as pl
from jax.experimental.pallas import tpu as pltpu
from jax.experimental.pallas import tpu_sc as plsc

N = 10000
E = 160000
NHOP = 40000
RN = 500
ALPHA = 0.2

NC = 2
NS = 16
NW = NC * NS

NPAD = 10240
RPAD = 512
EPAD = 163840
HPAD = 40960
CH = 64
D = 128
ROWS_PER_TILE = NPAD // NS
ET_N, EC_N = EPAD // NW, EPAD // NC
ET_H, EC_H = HPAD // NW, HPAD // NC
BT = 4096 // NW


def _leakyexp(z):
    return jnp.exp(-jnp.where(z >= 0, z, ALPHA * z))


def _zero_rows(buf, width, nrows):
    def body(i, c):
        for d in range(width // 16):
            buf[i, pl.ds(d * 16, 16)] = jnp.zeros((16,), jnp.float32)
        return c
    lax.fori_loop(0, nrows, body, 0)


def _init_accum(accum_n, accum_w, out_rows, w_rows, sid):
    _zero_rows(out_rows, D, CH)
    _zero_rows(w_rows, 16, CH)
    for j in range(ROWS_PER_TILE // CH):
        st = sid * ROWS_PER_TILE + j * CH
        pltpu.sync_copy(out_rows, accum_n.at[pl.ds(st, CH)])
        pltpu.sync_copy(w_rows, accum_w.at[pl.ds(st, CH)])


def _readout(accum_n, accum_w, out_rows, w_rows, out_n, out_w, cid, sid):
    for j in range(ROWS_PER_TILE // CH):
        st = sid * ROWS_PER_TILE + j * CH
        pltpu.sync_copy(accum_n.at[pl.ds(st, CH)], out_rows)
        pltpu.sync_copy(out_rows, out_n.at[pl.ds(cid * NPAD + st, CH)])
        pltpu.sync_copy(accum_w.at[pl.ds(st, CH)], w_rows)
        pltpu.sync_copy(w_rows, out_w.at[pl.ds(cid * NPAD + st, CH)])


def _att_body(nheads,
              dst_n, src_n, rt_n, dst_h, src_h, ra_h, rb_h, bidx_h,
              psrc_hbm, prel_hbm,
              tdst_hbm, tsrc_hbm, trel_hbm,
              out_n, out_w,
              accum_n, accum_w, dst_v, src_v, ra_v, rb_v,
              rows_s, rows_a, rows_b, out_rows, w_rows,
              dscal, sscal, rscal, rscalb,
              sem_a, sem_b, sem_c, sem_d, sem_e, sem_f, sem_g):
    cid = lax.axis_index("c")
    sid = lax.axis_index("s")
    lane = lax.iota(jnp.int32, 16)
    zeros16 = jnp.zeros((16,), jnp.int32)
    _init_accum(accum_n, accum_w, out_rows, w_rows, sid)
    plsc.subcore_barrier()

    if bidx_h is not None:
        def mrow(i, c):
            w_rows[i, pl.ds(0, 16)] = jnp.where(lane == 1, 1.0, 0.0)
            return c
        lax.fori_loop(0, CH, mrow, 0)
        wid = cid * NS + sid
        for j in range(BT // CH):
            pltpu.sync_copy(bidx_h.at[pl.ds(wid * BT + j * CH, CH)], dst_v)
            pltpu.sync_copy(w_rows, accum_w.at[dst_v], add=True)

    def scale_phase(nhop):
        def eb(e, c):
            sl = pl.ds(0, 16)
            z16 = dscal[e, sl] + sscal[e, sl] + rscal[e, sl]
            if nhop:
                z16 = z16 + rscalb[e, sl]
            w16 = _leakyexp(z16)
            w_rows[e, sl] = jnp.where(lane < nheads, w16, 0.0)
            esplat = jnp.full((16,), e, jnp.int32)
            w0 = plsc.load_gather(w_rows, [esplat, zeros16])
            if nheads == 2:
                w1 = plsc.load_gather(w_rows, [esplat, zeros16 + 1])
            for d in range(8):
                sld = pl.ds(d * 16, 16)
                v = rows_s[e, sld] + rows_a[e, sld]
                if nhop:
                    v = v + rows_b[e, sld]
                if nheads == 2:
                    out_rows[e, sld] = v * (w0 if d < 4 else w1)
                else:
                    out_rows[e, sld] = v * w0
            return c
        lax.fori_loop(0, CH, eb, 0)

    def nchunk(i, c):
        base = cid * EC_N + sid * ET_N + i * CH
        pltpu.sync_copy(dst_n.at[pl.ds(base, CH)], dst_v)
        pltpu.sync_copy(src_n.at[pl.ds(base, CH)], src_v)
        pltpu.sync_copy(rt_n.at[pl.ds(base, CH)], ra_v)
        cp1 = pltpu.async_copy(psrc_hbm.at[src_v], rows_s, sem_a)
        cp2 = pltpu.async_copy(prel_hbm.at[ra_v], rows_a, sem_b)
        cp3 = pltpu.async_copy(tdst_hbm.at[dst_v], dscal, sem_c)
        cp4 = pltpu.async_copy(tsrc_hbm.at[src_v], sscal, sem_d)
        cp5 = pltpu.async_copy(trel_hbm.at[ra_v], rscal, sem_e)
        cp1.wait()
        cp2.wait()
        cp3.wait()
        cp4.wait()
        cp5.wait()
        scale_phase(False)
        pltpu.sync_copy(out_rows, accum_n.at[dst_v], add=True)
        pltpu.sync_copy(w_rows, accum_w.at[dst_v], add=True)
        return c
    lax.fori_loop(0, ET_N // CH, nchunk, 0)

    def hchunk(i, c):
        gbase = cid * EC_H + sid * ET_H + i * CH
        pltpu.sync_copy(dst_h.at[pl.ds(gbase, CH)], dst_v)
        pltpu.sync_copy(src_h.at[pl.ds(gbase, CH)], src_v)
        pltpu.sync_copy(ra_h.at[pl.ds(gbase, CH)], ra_v)
        pltpu.sync_copy(rb_h.at[pl.ds(gbase, CH)], rb_v)
        cp1 = pltpu.async_copy(psrc_hbm.at[src_v], rows_s, sem_a)
        cp2 = pltpu.async_copy(prel_hbm.at[ra_v], rows_a, sem_b)
        cp3 = pltpu.async_copy(tdst_hbm.at[dst_v], dscal, sem_c)
        cp4 = pltpu.async_copy(tsrc_hbm.at[src_v], sscal, sem_d)
        cp5 = pltpu.async_copy(trel_hbm.at[ra_v], rscal, sem_e)
        cp6 = pltpu.async_copy(prel_hbm.at[rb_v], rows_b, sem_f)
        cp7 = pltpu.async_copy(trel_hbm.at[rb_v], rscalb, sem_g)
        cp1.wait()
        cp2.wait()
        cp3.wait()
        cp4.wait()
        cp5.wait()
        cp6.wait()
        cp7.wait()
        scale_phase(True)
        pltpu.sync_copy(out_rows, accum_n.at[dst_v], add=True)
        pltpu.sync_copy(w_rows, accum_w.at[dst_v], add=True)
        return c
    lax.fori_loop(0, ET_H // CH, hchunk, 0)

    plsc.subcore_barrier()
    _readout(accum_n, accum_w, out_rows, w_rows, out_n, out_w, cid, sid)


def _att1_body(dst_n, src_n, rt_n, dst_h, src_h, ra_h, rb_h,
               psrc_hbm, prel_hbm, tdst_hbm, tsrc_hbm, trel_hbm,
               out_n, out_w, *rest):
    _att_body(2, dst_n, src_n, rt_n, dst_h, src_h, ra_h, rb_h, None,
              psrc_hbm, prel_hbm, tdst_hbm, tsrc_hbm, trel_hbm,
              out_n, out_w, *rest)


def _att2_body(dst_n, src_n, rt_n, dst_h, src_h, ra_h, rb_h, bidx_h,
               psrc_hbm, prel_hbm, tdst_hbm, tsrc_hbm, trel_hbm,
               out_n, out_w, *rest):
    _att_body(1, dst_n, src_n, rt_n, dst_h, src_h, ra_h, rb_h, bidx_h,
              psrc_hbm, prel_hbm, tdst_hbm, tsrc_hbm, trel_hbm,
              out_n, out_w, *rest)


def _mesh():
    return plsc.VectorSubcoreMesh(core_axis_name="c", subcore_axis_name="s")


_OUT_TYPE = (jax.ShapeDtypeStruct((NC * NPAD, D), jnp.float32),
             jax.ShapeDtypeStruct((NC * NPAD, 16), jnp.float32))

_SCRATCH = [
    pltpu.VMEM_SHARED((NPAD, D), jnp.float32),
    pltpu.VMEM_SHARED((NPAD, 16), jnp.float32),
    pltpu.VMEM((CH,), jnp.int32),
    pltpu.VMEM((CH,), jnp.int32),
    pltpu.VMEM((CH,), jnp.int32),
    pltpu.VMEM((CH,), jnp.int32),
    pltpu.VMEM((CH, D), jnp.float32),
    pltpu.VMEM((CH, D), jnp.float32),
    pltpu.VMEM((CH, D), jnp.float32),
    pltpu.VMEM((CH, D), jnp.float32),
    pltpu.VMEM((CH, 16), jnp.float32),
    pltpu.VMEM((CH, 16), jnp.float32),
    pltpu.VMEM((CH, 16), jnp.float32),
    pltpu.VMEM((CH, 16), jnp.float32),
    pltpu.VMEM((CH, 16), jnp.float32),
    pltpu.SemaphoreType.DMA,
    pltpu.SemaphoreType.DMA,
    pltpu.SemaphoreType.DMA,
    pltpu.SemaphoreType.DMA,
    pltpu.SemaphoreType.DMA,
    pltpu.SemaphoreType.DMA,
    pltpu.SemaphoreType.DMA,
]

_att1 = pl.kernel(
    _att1_body, mesh=_mesh(),
    compiler_params=pltpu.CompilerParams(
        needs_layout_passes=False, use_tc_tiling_on_sc=False),
    out_type=_OUT_TYPE, scratch_types=_SCRATCH)

_att2 = pl.kernel(
    _att2_body, mesh=_mesh(),
    compiler_params=pltpu.CompilerParams(
        needs_layout_passes=False, use_tc_tiling_on_sc=False),
    out_type=_OUT_TYPE, scratch_types=_SCRATCH)


def _normalize_rows(x):
    n = jnp.linalg.norm(x, axis=1, keepdims=True)
    return x / jnp.maximum(n, 1e-12)


def _pad_to(x, n, value=0):
    return jnp.pad(x, [(0, n - x.shape[0])] + [(0, 0)] * (x.ndim - 1),
                   constant_values=value)


def _scal16(*cols):
    rows = cols[0].shape[0]
    out = jnp.zeros((rows, 16), jnp.float32)
    for i, c in enumerate(cols):
        out = out.at[:, i].set(c)
    return out


@jax.jit
def kernel(Corpus_, batch_inputs, edge_list, edge_type, train_indices_nhop,
           entity_embeddings, relation_embeddings, W_entities, W_spgat,
           a_heads, a2_heads, a_out, a2_out):
    f32 = jnp.float32
    ent = _normalize_rows(entity_embeddings)
    rel = _normalize_rows(relation_embeddings)
    rel_aug = jnp.concatenate([rel, jnp.zeros((RPAD - RN, 128), f32)], axis=0)

    i32 = jnp.int32
    dst_n = _pad_to(edge_list[0].astype(i32), EPAD, N)
    src_n = _pad_to(edge_list[1].astype(i32), EPAD, 0)
    rt_n = _pad_to(edge_type.astype(i32), EPAD, RN)
    tin = train_indices_nhop.astype(i32)
    dst_h = _pad_to(tin[:, 3], HPAD, N)
    src_h = _pad_to(tin[:, 0], HPAD, 0)
    ra_h = _pad_to(tin[:, 1], HPAD, RN)
    rb_h = _pad_to(tin[:, 2], HPAD, RN)
    bidx = batch_inputs[:, 2].astype(i32)

    p_dst = jnp.concatenate(
        [ent @ a_heads[0, :, :128].T, ent @ a_heads[1, :, :128].T], axis=1)
    p_src = jnp.concatenate(
        [ent @ a_heads[0, :, 128:256].T, ent @ a_heads[1, :, 128:256].T], axis=1)
    p_rel = jnp.concatenate(
        [rel_aug @ a_heads[0, :, 256:].T, rel_aug @ a_heads[1, :, 256:].T], axis=1)
    sd0 = p_dst[:, :64] @ a2_heads[0, 0]
    sd1 = p_dst[:, 64:] @ a2_heads[1, 0]
    ss0 = p_src[:, :64] @ a2_heads[0, 0]
    ss1 = p_src[:, 64:] @ a2_heads[1, 0]
    sr0 = p_rel[:, :64] @ a2_heads[0, 0]
    sr1 = p_rel[:, 64:] @ a2_heads[1, 0]
    tdst = _pad_to(_scal16(sd0, sd1), NPAD)
    tsrc = _pad_to(_scal16(ss0, ss1), NPAD)
    trel = _scal16(sr0, sr1)

    acc_n, acc_w = _att1(dst_n, src_n, rt_n, dst_h, src_h, ra_h, rb_h,
                         p_src, p_rel, tdst, tsrc, trel)
    acc_n = acc_n[:NPAD] + acc_n[NPAD:]
    acc_w = acc_w[:NPAD] + acc_w[NPAD:]
    r0 = acc_w[:N, 0:1]
    r1 = acc_w[:N, 1:2]
    h0 = (p_dst[:, :64] * r0 + acc_n[:N, :64]) / jnp.where(r0 == 0.0, 1e-12, r0)
    h1 = (p_dst[:, 64:] * r1 + acc_n[:N, 64:]) / jnp.where(r1 == 0.0, 1e-12, r1)
    x = jnp.concatenate([jax.nn.elu(h0), jax.nn.elu(h1)], axis=1)

    out_relation_1 = rel @ W_spgat
    orel_aug = jnp.concatenate(
        [out_relation_1, jnp.zeros((RPAD - RN, 128), f32)], axis=0)
    q_dst = x @ a_out[:, :128].T
    q_src = x @ a_out[:, 128:256].T
    q_rel = orel_aug @ a_out[:, 256:].T
    s2d = q_dst @ a2_out[0]
    s2s = q_src @ a2_out[0]
    s2r = q_rel @ a2_out[0]
    tdst2 = _pad_to(_scal16(s2d), NPAD)
    tsrc2 = _pad_to(_scal16(s2s), NPAD)
    trel2 = _scal16(s2r)

    acc2_n, acc2_w = _att2(dst_n, src_n, rt_n, dst_h, src_h, ra_h, rb_h, bidx,
                           q_src, q_rel, tdst2, tsrc2, trel2)
    acc2_n = acc2_n[:NPAD] + acc2_n[NPAD:]
    acc2_w = acc2_w[:NPAD] + acc2_w[NPAD:]
    r2 = acc2_w[:N, 0:1]
    h2 = (q_dst * r2 + acc2_n[:N]) / jnp.where(r2 == 0.0, 1e-12, r2)
    x2 = jax.nn.elu(h2)
    mask = (acc2_w[:N, 1:2] > 0.0).astype(f32)

    out_entity_1 = _normalize_rows(ent @ W_entities + mask * x2)
    return (out_entity_1, out_relation_1)


# speedup vs baseline: 2.5644x; 1.2584x over previous
import jax
import jax.numpy as jnp
from jax import lax
from jax.experimental import pallas as pl
from jax.experimental.pallas import tpu as pltpu
from jax.experimental.pallas import tpu_sc as plsc

N = 10000
E = 160000
NHOP = 40000
RN = 500
ALPHA = 0.2

NC = 2
NS = 16
NW = NC * NS

NPAD = 10240
RPAD = 512
EPAD = 163840
HPAD = 40960
CH = 32
D = 128
ROWS_PER_TILE = NPAD // NS
ET_N, EC_N = EPAD // NW, EPAD // NC
ET_H, EC_H = HPAD // NW, HPAD // NC
BT = 4096 // NW


def _leakyexp(z):
    return jnp.exp(-jnp.where(z >= 0, z, ALPHA * z))


def _zero_rows(buf, width, nrows):
    def body(i, c):
        for d in range(width // 16):
            buf[i, pl.ds(d * 16, 16)] = jnp.zeros((16,), jnp.float32)
        return c
    lax.fori_loop(0, nrows, body, 0)


def _att_body(nheads,
              dst_n, src_n, rt_n, dst_h, src_h, ra_h, rb_h, bidx_h,
              psrc_hbm, prel_hbm, tdst_hbm, tsrc_hbm, trel_hbm,
              out_n, out_w,
              accum_n, accum_w,
              dst_v0, dst_v1, src_v0, src_v1, ra_v0, ra_v1, rb_v0, rb_v1,
              rows_s0, rows_s1, rows_a0, rows_a1, rows_b0, rows_b1,
              dscal0, dscal1, sscal0, sscal1, rscal0, rscal1,
              rscalb0, rscalb1,
              out_rows, w_rows, sem0, sem1):
    cid = lax.axis_index("c")
    sid = lax.axis_index("s")
    lane = lax.iota(jnp.int32, 16)
    zeros16 = jnp.zeros((16,), jnp.int32)
    dst_v = [dst_v0, dst_v1]
    src_v = [src_v0, src_v1]
    ra_v = [ra_v0, ra_v1]
    rb_v = [rb_v0, rb_v1]
    rows_s = [rows_s0, rows_s1]
    rows_a = [rows_a0, rows_a1]
    rows_b = [rows_b0, rows_b1]
    dscal = [dscal0, dscal1]
    sscal = [sscal0, sscal1]
    rscal = [rscal0, rscal1]
    rscalb = [rscalb0, rscalb1]
    sem = [sem0, sem1]

    _zero_rows(out_rows, D, CH)
    _zero_rows(w_rows, 16, CH)
    for j in range(ROWS_PER_TILE // CH):
        st = sid * ROWS_PER_TILE + j * CH
        pltpu.sync_copy(out_rows, accum_n.at[pl.ds(st, CH)])
        pltpu.sync_copy(w_rows, accum_w.at[pl.ds(st, CH)])
    plsc.subcore_barrier()

    if bidx_h is not None:
        def mrow(i, c):
            w_rows[i, pl.ds(0, 16)] = jnp.where(lane == 1, 1.0, 0.0)
            return c
        lax.fori_loop(0, CH, mrow, 0)
        wid = cid * NS + sid
        for j in range(BT // CH):
            pltpu.sync_copy(bidx_h.at[pl.ds(wid * BT + j * CH, CH)], dst_v[0])
            pltpu.sync_copy(w_rows, accum_w.at[dst_v[0]], add=True)

    def issue(b, base, nhop):
        if nhop:
            pltpu.sync_copy(dst_h.at[pl.ds(base, CH)], dst_v[b])
            pltpu.sync_copy(src_h.at[pl.ds(base, CH)], src_v[b])
            pltpu.sync_copy(ra_h.at[pl.ds(base, CH)], ra_v[b])
            pltpu.sync_copy(rb_h.at[pl.ds(base, CH)], rb_v[b])
        else:
            pltpu.sync_copy(dst_n.at[pl.ds(base, CH)], dst_v[b])
            pltpu.sync_copy(src_n.at[pl.ds(base, CH)], src_v[b])
            pltpu.sync_copy(rt_n.at[pl.ds(base, CH)], ra_v[b])
        pltpu.async_copy(psrc_hbm.at[src_v[b]], rows_s[b], sem[b])
        pltpu.async_copy(prel_hbm.at[ra_v[b]], rows_a[b], sem[b])
        pltpu.async_copy(tdst_hbm.at[dst_v[b]], dscal[b], sem[b])
        pltpu.async_copy(tsrc_hbm.at[src_v[b]], sscal[b], sem[b])
        pltpu.async_copy(trel_hbm.at[ra_v[b]], rscal[b], sem[b])
        if nhop:
            pltpu.async_copy(prel_hbm.at[rb_v[b]], rows_b[b], sem[b])
            pltpu.async_copy(trel_hbm.at[rb_v[b]], rscalb[b], sem[b])

    def drain(b, nhop):
        pltpu.make_async_copy(psrc_hbm.at[src_v[b]], rows_s[b], sem[b]).wait()
        pltpu.make_async_copy(prel_hbm.at[ra_v[b]], rows_a[b], sem[b]).wait()
        pltpu.make_async_copy(tdst_hbm.at[dst_v[b]], dscal[b], sem[b]).wait()
        pltpu.make_async_copy(tsrc_hbm.at[src_v[b]], sscal[b], sem[b]).wait()
        pltpu.make_async_copy(trel_hbm.at[ra_v[b]], rscal[b], sem[b]).wait()
        if nhop:
            pltpu.make_async_copy(prel_hbm.at[rb_v[b]], rows_b[b],
                                  sem[b]).wait()
            pltpu.make_async_copy(trel_hbm.at[rb_v[b]], rscalb[b],
                                  sem[b]).wait()

    def compute(b, nhop):
        def eb(e, c):
            sl = pl.ds(0, 16)
            z16 = dscal[b][e, sl] + sscal[b][e, sl] + rscal[b][e, sl]
            if nhop:
                z16 = z16 + rscalb[b][e, sl]
            w16 = _leakyexp(z16)
            w_rows[e, sl] = jnp.where(lane < nheads, w16, 0.0)
            esplat = jnp.full((16,), e, jnp.int32)
            w0 = plsc.load_gather(w_rows, [esplat, zeros16])
            if nheads == 2:
                w1 = plsc.load_gather(w_rows, [esplat, zeros16 + 1])
            for d in range(8):
                sld = pl.ds(d * 16, 16)
                v = rows_s[b][e, sld] + rows_a[b][e, sld]
                if nhop:
                    v = v + rows_b[b][e, sld]
                if nheads == 2:
                    out_rows[e, sld] = v * (w0 if d < 4 else w1)
                else:
                    out_rows[e, sld] = v * w0
            return c
        lax.fori_loop(0, CH, eb, 0)
        pltpu.sync_copy(out_rows, accum_n.at[dst_v[b]], add=True)
        pltpu.sync_copy(w_rows, accum_w.at[dst_v[b]], add=True)

    def edge_loop(nchunks, cbase, tchunks, nhop):
        issue(0, cbase, nhop)

        def pair(i2, c):
            base = cbase + i2 * (2 * CH)
            drain(0, nhop)
            issue(1, base + CH, nhop)
            compute(0, nhop)
            drain(1, nhop)

            @pl.when(i2 + 1 < nchunks // 2)
            def _():
                issue(0, base + 2 * CH, nhop)
            compute(1, nhop)
            return c
        lax.fori_loop(0, nchunks // 2, pair, 0)

    edge_loop(ET_N // CH, cid * EC_N + sid * ET_N, ET_N // CH, False)
    edge_loop(ET_H // CH, cid * EC_H + sid * ET_H, ET_H // CH, True)

    plsc.subcore_barrier()
    for j in range(ROWS_PER_TILE // CH):
        st = sid * ROWS_PER_TILE + j * CH
        pltpu.sync_copy(accum_n.at[pl.ds(st, CH)], out_rows)
        pltpu.sync_copy(out_rows, out_n.at[pl.ds(cid * NPAD + st, CH)])
        pltpu.sync_copy(accum_w.at[pl.ds(st, CH)], w_rows)
        pltpu.sync_copy(w_rows, out_w.at[pl.ds(cid * NPAD + st, CH)])


def _att1_body(dst_n, src_n, rt_n, dst_h, src_h, ra_h, rb_h,
               psrc_hbm, prel_hbm, tdst_hbm, tsrc_hbm, trel_hbm,
               out_n, out_w, *rest):
    _att_body(2, dst_n, src_n, rt_n, dst_h, src_h, ra_h, rb_h, None,
              psrc_hbm, prel_hbm, tdst_hbm, tsrc_hbm, trel_hbm,
              out_n, out_w, *rest)


def _att2_body(dst_n, src_n, rt_n, dst_h, src_h, ra_h, rb_h, bidx_h,
               psrc_hbm, prel_hbm, tdst_hbm, tsrc_hbm, trel_hbm,
               out_n, out_w, *rest):
    _att_body(1, dst_n, src_n, rt_n, dst_h, src_h, ra_h, rb_h, bidx_h,
              psrc_hbm, prel_hbm, tdst_hbm, tsrc_hbm, trel_hbm,
              out_n, out_w, *rest)


def _mesh():
    return plsc.VectorSubcoreMesh(core_axis_name="c", subcore_axis_name="s")


_OUT_TYPE = (jax.ShapeDtypeStruct((NC * NPAD, D), jnp.float32),
             jax.ShapeDtypeStruct((NC * NPAD, 16), jnp.float32))

_SCRATCH = (
    [pltpu.VMEM_SHARED((NPAD, D), jnp.float32),
     pltpu.VMEM_SHARED((NPAD, 16), jnp.float32)]
    + [pltpu.VMEM((CH,), jnp.int32)] * 8
    + [pltpu.VMEM((CH, D), jnp.float32)] * 6
    + [pltpu.VMEM((CH, 16), jnp.float32)] * 8
    + [pltpu.VMEM((CH, D), jnp.float32),
       pltpu.VMEM((CH, 16), jnp.float32),
       pltpu.SemaphoreType.DMA,
       pltpu.SemaphoreType.DMA]
)

_att1 = pl.kernel(
    _att1_body, mesh=_mesh(),
    compiler_params=pltpu.CompilerParams(
        needs_layout_passes=False, use_tc_tiling_on_sc=False),
    out_type=_OUT_TYPE, scratch_types=_SCRATCH)

_att2 = pl.kernel(
    _att2_body, mesh=_mesh(),
    compiler_params=pltpu.CompilerParams(
        needs_layout_passes=False, use_tc_tiling_on_sc=False),
    out_type=_OUT_TYPE, scratch_types=_SCRATCH)


def _normalize_rows(x):
    n = jnp.linalg.norm(x, axis=1, keepdims=True)
    return x / jnp.maximum(n, 1e-12)


def _pad_to(x, n, value=0):
    return jnp.pad(x, [(0, n - x.shape[0])] + [(0, 0)] * (x.ndim - 1),
                   constant_values=value)


def _scal16(*cols):
    rows = cols[0].shape[0]
    out = jnp.zeros((rows, 16), jnp.float32)
    for i, c in enumerate(cols):
        out = out.at[:, i].set(c)
    return out


@jax.jit
def kernel(Corpus_, batch_inputs, edge_list, edge_type, train_indices_nhop,
           entity_embeddings, relation_embeddings, W_entities, W_spgat,
           a_heads, a2_heads, a_out, a2_out):
    f32 = jnp.float32
    ent = _normalize_rows(entity_embeddings)
    rel = _normalize_rows(relation_embeddings)
    rel_aug = jnp.concatenate([rel, jnp.zeros((RPAD - RN, 128), f32)], axis=0)

    i32 = jnp.int32
    dst_n = _pad_to(edge_list[0].astype(i32), EPAD, N)
    src_n = _pad_to(edge_list[1].astype(i32), EPAD, 0)
    rt_n = _pad_to(edge_type.astype(i32), EPAD, RN)
    tin = train_indices_nhop.astype(i32)
    dst_h = _pad_to(tin[:, 3], HPAD, N)
    src_h = _pad_to(tin[:, 0], HPAD, 0)
    ra_h = _pad_to(tin[:, 1], HPAD, RN)
    rb_h = _pad_to(tin[:, 2], HPAD, RN)
    bidx = batch_inputs[:, 2].astype(i32)

    p_dst = jnp.concatenate(
        [ent @ a_heads[0, :, :128].T, ent @ a_heads[1, :, :128].T], axis=1)
    p_src = jnp.concatenate(
        [ent @ a_heads[0, :, 128:256].T, ent @ a_heads[1, :, 128:256].T], axis=1)
    p_rel = jnp.concatenate(
        [rel_aug @ a_heads[0, :, 256:].T, rel_aug @ a_heads[1, :, 256:].T], axis=1)
    sd0 = p_dst[:, :64] @ a2_heads[0, 0]
    sd1 = p_dst[:, 64:] @ a2_heads[1, 0]
    ss0 = p_src[:, :64] @ a2_heads[0, 0]
    ss1 = p_src[:, 64:] @ a2_heads[1, 0]
    sr0 = p_rel[:, :64] @ a2_heads[0, 0]
    sr1 = p_rel[:, 64:] @ a2_heads[1, 0]
    tdst = _pad_to(_scal16(sd0, sd1), NPAD)
    tsrc = _pad_to(_scal16(ss0, ss1), NPAD)
    trel = _scal16(sr0, sr1)

    acc_n, acc_w = _att1(dst_n, src_n, rt_n, dst_h, src_h, ra_h, rb_h,
                         p_src, p_rel, tdst, tsrc, trel)
    acc_n = acc_n[:NPAD] + acc_n[NPAD:]
    acc_w = acc_w[:NPAD] + acc_w[NPAD:]
    r0 = acc_w[:N, 0:1]
    r1 = acc_w[:N, 1:2]
    h0 = (p_dst[:, :64] * r0 + acc_n[:N, :64]) / jnp.where(r0 == 0.0, 1e-12, r0)
    h1 = (p_dst[:, 64:] * r1 + acc_n[:N, 64:]) / jnp.where(r1 == 0.0, 1e-12, r1)
    x = jnp.concatenate([jax.nn.elu(h0), jax.nn.elu(h1)], axis=1)

    out_relation_1 = rel @ W_spgat
    orel_aug = jnp.concatenate(
        [out_relation_1, jnp.zeros((RPAD - RN, 128), f32)], axis=0)
    q_dst = x @ a_out[:, :128].T
    q_src = x @ a_out[:, 128:256].T
    q_rel = orel_aug @ a_out[:, 256:].T
    s2d = q_dst @ a2_out[0]
    s2s = q_src @ a2_out[0]
    s2r = q_rel @ a2_out[0]
    tdst2 = _pad_to(_scal16(s2d), NPAD)
    tsrc2 = _pad_to(_scal16(s2s), NPAD)
    trel2 = _scal16(s2r)

    acc2_n, acc2_w = _att2(dst_n, src_n, rt_n, dst_h, src_h, ra_h, rb_h, bidx,
                           q_src, q_rel, tdst2, tsrc2, trel2)
    acc2_n = acc2_n[:NPAD] + acc2_n[NPAD:]
    acc2_w = acc2_w[:NPAD] + acc2_w[NPAD:]
    r2 = acc2_w[:N, 0:1]
    h2 = (q_dst * r2 + acc2_n[:N]) / jnp.where(r2 == 0.0, 1e-12, r2)
    x2 = jax.nn.elu(h2)
    mask = (acc2_w[:N, 1:2] > 0.0).astype(f32)

    out_entity_1 = _normalize_rows(ent @ W_entities + mask * x2)
    return (out_entity_1, out_relation_1)


# speedup vs baseline: 3.4701x; 1.3532x over previous
import jax
import jax.numpy as jnp
from jax import lax
from jax.experimental import pallas as pl
from jax.experimental.pallas import tpu as pltpu
from jax.experimental.pallas import tpu_sc as plsc

N = 10000
E = 160000
NHOP = 40000
RN = 500
ALPHA = 0.2

NC = 2
NS = 16
NW = NC * NS

NPAD = 10240
RPAD = 512
EPAD = 163840
HPAD = 40960
CH = 32
D = 128
ROWS_PER_TILE = NPAD // NS
ET_N, EC_N = EPAD // NW, EPAD // NC
ET_H, EC_H = HPAD // NW, HPAD // NC
BT = 4096 // NW


def _leakyexp(z):
    return jnp.exp(-jnp.where(z >= 0, z, ALPHA * z))


def _zero_rows(buf, width, nrows):
    def body(i, c):
        for d in range(width // 16):
            buf[i, pl.ds(d * 16, 16)] = jnp.zeros((16,), jnp.float32)
        return c
    lax.fori_loop(0, nrows, body, 0)


BLK = 4


def _att_body(nheads,
              dst_n, src_n, rt_n, dst_h, src_h, ra_h, rb_h, bidx_h,
              psrc_hbm, prel_hbm, tdst_hbm, tsrc_hbm, trel_hbm,
              out_n, out_w,
              accum_n, accum_w,
              dstblk0, dstblk1, srcblk0, srcblk1, rablk0, rablk1,
              rbblk0, rbblk1, dst_v0, dst_v1,
              rows_s0, rows_s1, rows_a0, rows_a1, rows_b0, rows_b1,
              dscal0, dscal1, sscal0, sscal1, rscal0, rscal1,
              rscalb0, rscalb1,
              out_rows0, out_rows1, w_rows,
              sem0, sem1, semi0, semi1, semw0, semw1):
    cid = lax.axis_index("c")
    sid = lax.axis_index("s")
    lane = lax.iota(jnp.int32, 16)
    zeros16 = jnp.zeros((16,), jnp.int32)
    dstblk = [dstblk0, dstblk1]
    srcblk = [srcblk0, srcblk1]
    rablk = [rablk0, rablk1]
    rbblk = [rbblk0, rbblk1]
    dst_v = [dst_v0, dst_v1]
    rows_s = [rows_s0, rows_s1]
    rows_a = [rows_a0, rows_a1]
    rows_b = [rows_b0, rows_b1]
    dscal = [dscal0, dscal1]
    sscal = [sscal0, sscal1]
    rscal = [rscal0, rscal1]
    rscalb = [rscalb0, rscalb1]
    out_rows = [out_rows0, out_rows1]
    sem = [sem0, sem1]
    semi = [semi0, semi1]
    semw = [semw0, semw1]
    SPAN = BLK * CH

    _zero_rows(out_rows[0], D, CH)
    _zero_rows(w_rows, 16, CH)
    for j in range(ROWS_PER_TILE // CH):
        st = sid * ROWS_PER_TILE + j * CH
        pltpu.sync_copy(out_rows[0], accum_n.at[pl.ds(st, CH)])
        pltpu.sync_copy(w_rows, accum_w.at[pl.ds(st, CH)])
    plsc.subcore_barrier()

    if bidx_h is not None:
        def mrow(i, c):
            w_rows[i, pl.ds(0, 16)] = jnp.where(lane == 1, 1.0, 0.0)
            return c
        lax.fori_loop(0, CH, mrow, 0)
        wid = cid * NS + sid
        for j in range(BT // CH):
            pltpu.sync_copy(bidx_h.at[pl.ds(wid * BT + j * CH, CH)], dst_v[0])
            pltpu.sync_copy(w_rows, accum_w.at[dst_v[0]], add=True)

    def idx_arrays(nhop):
        return ([dst_h, src_h, ra_h, rb_h] if nhop
                else [dst_n, src_n, rt_n])

    def idx_bufs(p, nhop):
        return ([dstblk[p], srcblk[p], rablk[p], rbblk[p]] if nhop
                else [dstblk[p], srcblk[p], rablk[p]])

    def issue_idx(p, brow, nhop):
        for arr, buf in zip(idx_arrays(nhop), idx_bufs(p, nhop)):
            pltpu.async_copy(arr.at[pl.ds(brow, BLK)], buf, semi[p])

    def drain_idx(p, brow, nhop):
        for arr, buf in zip(idx_arrays(nhop), idx_bufs(p, nhop)):
            pltpu.make_async_copy(arr.at[pl.ds(brow, BLK)], buf,
                                  semi[p]).wait()

    def gather_list(b, p, k, nhop):
        pairs = [
            (psrc_hbm.at[srcblk[p].at[k]], rows_s[b]),
            (prel_hbm.at[rablk[p].at[k]], rows_a[b]),
            (tdst_hbm.at[dstblk[p].at[k]], dscal[b]),
            (tsrc_hbm.at[srcblk[p].at[k]], sscal[b]),
            (trel_hbm.at[rablk[p].at[k]], rscal[b]),
        ]
        if nhop:
            pairs += [(prel_hbm.at[rbblk[p].at[k]], rows_b[b]),
                      (trel_hbm.at[rbblk[p].at[k]], rscalb[b])]
        return pairs

    def issue(b, p, k, nhop):
        for s, d in gather_list(b, p, k, nhop):
            pltpu.async_copy(s, d, sem[b])

    def drain(b, p, k, nhop):
        for s, d in gather_list(b, p, k, nhop):
            pltpu.make_async_copy(s, d, sem[b]).wait()

    def drain_scatter(b):
        pltpu.make_async_copy(out_rows[b], accum_n.at[dst_v[b]],
                              semw[b]).wait()

    def compute(b, p, k, nhop):
        for g in range(CH // 16):
            dst_v[b][pl.ds(g * 16, 16)] = dstblk[p][k, pl.ds(g * 16, 16)]

        def eb(e, c):
            sl = pl.ds(0, 16)
            z16 = dscal[b][e, sl] + sscal[b][e, sl] + rscal[b][e, sl]
            if nhop:
                z16 = z16 + rscalb[b][e, sl]
            w16 = _leakyexp(z16)
            w_rows[e, sl] = jnp.where(lane < nheads, w16, 0.0)
            esplat = jnp.full((16,), e, jnp.int32)
            w0 = plsc.load_gather(w_rows, [esplat, zeros16])
            if nheads == 2:
                w1 = plsc.load_gather(w_rows, [esplat, zeros16 + 1])
            for d in range(8):
                sld = pl.ds(d * 16, 16)
                v = rows_s[b][e, sld] + rows_a[b][e, sld]
                if nhop:
                    v = v + rows_b[b][e, sld]
                if nheads == 2:
                    out_rows[b][e, sld] = v * (w0 if d < 4 else w1)
                else:
                    out_rows[b][e, sld] = v * w0
            return c
        lax.fori_loop(0, CH, eb, 0)
        pltpu.async_copy(out_rows[b], accum_n.at[dst_v[b]], semw[b], add=True)
        pltpu.sync_copy(w_rows, accum_w.at[dst_v[b]], add=True)

    def edge_loop(nchunks, cbase, nhop, first_loop):
        nblocks = nchunks // BLK
        crow = cbase // CH
        issue_idx(0, crow, nhop)
        drain_idx(0, crow, nhop)
        issue(0, 0, 0, nhop)
        issue_idx(1, crow + BLK, nhop)

        def block_pair(i2, c):
            for parity in range(2):
                ib = i2 * 2 + parity
                brow = crow + ib * BLK
                for k in range(BLK):
                    b = k % 2
                    drain(b, parity, k, nhop)
                    if k + 1 < BLK:
                        issue(1 - b, parity, k + 1, nhop)
                    else:
                        @pl.when(ib + 1 < nblocks)
                        def _():
                            drain_idx(1 - parity, brow + BLK, nhop)
                            issue(1 - b, 1 - parity, 0, nhop)
                    if first_loop:
                        @pl.when(ib * BLK + k >= 2)
                        def _():
                            drain_scatter(b)
                    else:
                        drain_scatter(b)
                    compute(b, parity, k, nhop)

                @pl.when(ib + 2 < nblocks)
                def _():
                    issue_idx(parity, brow + 2 * BLK, nhop)
            return c
        lax.fori_loop(0, nblocks // 2, block_pair, 0)

    edge_loop(ET_N // CH, cid * EC_N + sid * ET_N, False, True)
    edge_loop(ET_H // CH, cid * EC_H + sid * ET_H, True, False)
    drain_scatter(0)
    drain_scatter(1)

    plsc.subcore_barrier()
    for j in range(ROWS_PER_TILE // CH):
        st = sid * ROWS_PER_TILE + j * CH
        pltpu.sync_copy(accum_n.at[pl.ds(st, CH)], out_rows[0])
        pltpu.sync_copy(out_rows[0], out_n.at[pl.ds(cid * NPAD + st, CH)])
        pltpu.sync_copy(accum_w.at[pl.ds(st, CH)], w_rows)
        pltpu.sync_copy(w_rows, out_w.at[pl.ds(cid * NPAD + st, CH)])


def _att1_body(dst_n, src_n, rt_n, dst_h, src_h, ra_h, rb_h,
               psrc_hbm, prel_hbm, tdst_hbm, tsrc_hbm, trel_hbm,
               out_n, out_w, *rest):
    _att_body(2, dst_n, src_n, rt_n, dst_h, src_h, ra_h, rb_h, None,
              psrc_hbm, prel_hbm, tdst_hbm, tsrc_hbm, trel_hbm,
              out_n, out_w, *rest)


def _att2_body(dst_n, src_n, rt_n, dst_h, src_h, ra_h, rb_h, bidx_h,
               psrc_hbm, prel_hbm, tdst_hbm, tsrc_hbm, trel_hbm,
               out_n, out_w, *rest):
    _att_body(1, dst_n, src_n, rt_n, dst_h, src_h, ra_h, rb_h, bidx_h,
              psrc_hbm, prel_hbm, tdst_hbm, tsrc_hbm, trel_hbm,
              out_n, out_w, *rest)


def _mesh():
    return plsc.VectorSubcoreMesh(core_axis_name="c", subcore_axis_name="s")


_OUT_TYPE = (jax.ShapeDtypeStruct((NC * NPAD, D), jnp.float32),
             jax.ShapeDtypeStruct((NC * NPAD, 16), jnp.float32))

_SCRATCH = (
    [pltpu.VMEM_SHARED((NPAD, D), jnp.float32),
     pltpu.VMEM_SHARED((NPAD, 16), jnp.float32)]
    + [pltpu.VMEM((BLK, CH), jnp.int32)] * 8
    + [pltpu.VMEM((CH,), jnp.int32)] * 2
    + [pltpu.VMEM((CH, D), jnp.float32)] * 6
    + [pltpu.VMEM((CH, 16), jnp.float32)] * 8
    + [pltpu.VMEM((CH, D), jnp.float32)] * 2
    + [pltpu.VMEM((CH, 16), jnp.float32)]
    + [pltpu.SemaphoreType.DMA] * 6
)

_att1 = pl.kernel(
    _att1_body, mesh=_mesh(),
    compiler_params=pltpu.CompilerParams(
        needs_layout_passes=False, use_tc_tiling_on_sc=False),
    out_type=_OUT_TYPE, scratch_types=_SCRATCH)

_att2 = pl.kernel(
    _att2_body, mesh=_mesh(),
    compiler_params=pltpu.CompilerParams(
        needs_layout_passes=False, use_tc_tiling_on_sc=False),
    out_type=_OUT_TYPE, scratch_types=_SCRATCH)


def _normalize_rows(x):
    n = jnp.linalg.norm(x, axis=1, keepdims=True)
    return x / jnp.maximum(n, 1e-12)


def _pad_to(x, n, value=0):
    return jnp.pad(x, [(0, n - x.shape[0])] + [(0, 0)] * (x.ndim - 1),
                   constant_values=value)


def _scal16(*cols):
    rows = cols[0].shape[0]
    out = jnp.zeros((rows, 16), jnp.float32)
    for i, c in enumerate(cols):
        out = out.at[:, i].set(c)
    return out


@jax.jit
def kernel(Corpus_, batch_inputs, edge_list, edge_type, train_indices_nhop,
           entity_embeddings, relation_embeddings, W_entities, W_spgat,
           a_heads, a2_heads, a_out, a2_out):
    f32 = jnp.float32
    ent = _normalize_rows(entity_embeddings)
    rel = _normalize_rows(relation_embeddings)
    rel_aug = jnp.concatenate([rel, jnp.zeros((RPAD - RN, 128), f32)], axis=0)

    i32 = jnp.int32
    dst_n = _pad_to(edge_list[0].astype(i32), EPAD, N).reshape(-1, CH)
    src_n = _pad_to(edge_list[1].astype(i32), EPAD, 0).reshape(-1, CH)
    rt_n = _pad_to(edge_type.astype(i32), EPAD, RN).reshape(-1, CH)
    tin = train_indices_nhop.astype(i32)
    dst_h = _pad_to(tin[:, 3], HPAD, N).reshape(-1, CH)
    src_h = _pad_to(tin[:, 0], HPAD, 0).reshape(-1, CH)
    ra_h = _pad_to(tin[:, 1], HPAD, RN).reshape(-1, CH)
    rb_h = _pad_to(tin[:, 2], HPAD, RN).reshape(-1, CH)
    bidx = batch_inputs[:, 2].astype(i32)

    p_dst = jnp.concatenate(
        [ent @ a_heads[0, :, :128].T, ent @ a_heads[1, :, :128].T], axis=1)
    p_src = jnp.concatenate(
        [ent @ a_heads[0, :, 128:256].T, ent @ a_heads[1, :, 128:256].T], axis=1)
    p_rel = jnp.concatenate(
        [rel_aug @ a_heads[0, :, 256:].T, rel_aug @ a_heads[1, :, 256:].T], axis=1)
    sd0 = p_dst[:, :64] @ a2_heads[0, 0]
    sd1 = p_dst[:, 64:] @ a2_heads[1, 0]
    ss0 = p_src[:, :64] @ a2_heads[0, 0]
    ss1 = p_src[:, 64:] @ a2_heads[1, 0]
    sr0 = p_rel[:, :64] @ a2_heads[0, 0]
    sr1 = p_rel[:, 64:] @ a2_heads[1, 0]
    tdst = _pad_to(_scal16(sd0, sd1), NPAD)
    tsrc = _pad_to(_scal16(ss0, ss1), NPAD)
    trel = _scal16(sr0, sr1)

    acc_n, acc_w = _att1(dst_n, src_n, rt_n, dst_h, src_h, ra_h, rb_h,
                         p_src, p_rel, tdst, tsrc, trel)
    acc_n = acc_n[:NPAD] + acc_n[NPAD:]
    acc_w = acc_w[:NPAD] + acc_w[NPAD:]
    r0 = acc_w[:N, 0:1]
    r1 = acc_w[:N, 1:2]
    h0 = (p_dst[:, :64] * r0 + acc_n[:N, :64]) / jnp.where(r0 == 0.0, 1e-12, r0)
    h1 = (p_dst[:, 64:] * r1 + acc_n[:N, 64:]) / jnp.where(r1 == 0.0, 1e-12, r1)
    x = jnp.concatenate([jax.nn.elu(h0), jax.nn.elu(h1)], axis=1)

    out_relation_1 = rel @ W_spgat
    orel_aug = jnp.concatenate(
        [out_relation_1, jnp.zeros((RPAD - RN, 128), f32)], axis=0)
    q_dst = x @ a_out[:, :128].T
    q_src = x @ a_out[:, 128:256].T
    q_rel = orel_aug @ a_out[:, 256:].T
    s2d = q_dst @ a2_out[0]
    s2s = q_src @ a2_out[0]
    s2r = q_rel @ a2_out[0]
    tdst2 = _pad_to(_scal16(s2d), NPAD)
    tsrc2 = _pad_to(_scal16(s2s), NPAD)
    trel2 = _scal16(s2r)

    acc2_n, acc2_w = _att2(dst_n, src_n, rt_n, dst_h, src_h, ra_h, rb_h, bidx,
                           q_src, q_rel, tdst2, tsrc2, trel2)
    acc2_n = acc2_n[:NPAD] + acc2_n[NPAD:]
    acc2_w = acc2_w[:NPAD] + acc2_w[NPAD:]
    r2 = acc2_w[:N, 0:1]
    h2 = (q_dst * r2 + acc2_n[:N]) / jnp.where(r2 == 0.0, 1e-12, r2)
    x2 = jax.nn.elu(h2)
    mask = (acc2_w[:N, 1:2] > 0.0).astype(f32)

    out_entity_1 = _normalize_rows(ent @ W_entities + mask * x2)
    return (out_entity_1, out_relation_1)


# speedup vs baseline: 3.4941x; 1.0069x over previous
import jax
import jax.numpy as jnp
from jax import lax
from jax.experimental import pallas as pl
from jax.experimental.pallas import tpu as pltpu
from jax.experimental.pallas import tpu_sc as plsc

N = 10000
E = 160000
NHOP = 40000
RN = 500
ALPHA = 0.2

NC = 2
NS = 16
NW = NC * NS

NPAD = 10240
RPAD = 512
EPAD = 163840
HPAD = 40960
CH = 32
D = 128
ROWS_PER_TILE = NPAD // NS
ET_N, EC_N = EPAD // NW, EPAD // NC
ET_H, EC_H = HPAD // NW, HPAD // NC
BT = 4096 // NW


def _leakyexp(z):
    return jnp.exp(-jnp.where(z >= 0, z, ALPHA * z))


def _zero_rows(buf, width, nrows):
    def body(i, c):
        for d in range(width // 16):
            buf[i, pl.ds(d * 16, 16)] = jnp.zeros((16,), jnp.float32)
        return c
    lax.fori_loop(0, nrows, body, 0)


BLK = 2


def _att_body(nheads,
              dst_n, src_n, rt_n, dst_h, src_h, ra_h, rb_h, bidx_h,
              psrc_hbm, prel_hbm, tdst_hbm, tsrc_hbm, trel_hbm,
              out_n, out_w,
              accum_n, accum_w,
              dstblk0, dstblk1, srcblk0, srcblk1, rablk0, rablk1,
              rbblk0, rbblk1, dst_v0, dst_v1,
              rows_s0, rows_s1, rows_a0, rows_a1, rows_b0, rows_b1,
              dscal0, dscal1, sscal0, sscal1, rscal0, rscal1,
              rscalb0, rscalb1,
              out_rows0, out_rows1, w_rows0, w_rows1,
              sem0, sem1, semi0, semi1, semw0, semw1):
    cid = lax.axis_index("c")
    sid = lax.axis_index("s")
    lane = lax.iota(jnp.int32, 16)
    zeros16 = jnp.zeros((16,), jnp.int32)
    dstblk = [dstblk0, dstblk1]
    srcblk = [srcblk0, srcblk1]
    rablk = [rablk0, rablk1]
    rbblk = [rbblk0, rbblk1]
    dst_v = [dst_v0, dst_v1]
    rows_s = [rows_s0, rows_s1]
    rows_a = [rows_a0, rows_a1]
    rows_b = [rows_b0, rows_b1]
    dscal = [dscal0, dscal1]
    sscal = [sscal0, sscal1]
    rscal = [rscal0, rscal1]
    rscalb = [rscalb0, rscalb1]
    out_rows = [out_rows0, out_rows1]
    w_rows = [w_rows0, w_rows1]
    sem = [sem0, sem1]
    semi = [semi0, semi1]
    semw = [semw0, semw1]
    SPAN = BLK * CH

    _zero_rows(out_rows[0], D, CH)
    _zero_rows(w_rows[0], 16, CH)
    for j in range(ROWS_PER_TILE // CH):
        st = sid * ROWS_PER_TILE + j * CH
        pltpu.sync_copy(out_rows[0], accum_n.at[pl.ds(st, CH)])
        pltpu.sync_copy(w_rows[0], accum_w.at[pl.ds(st, CH)])
    plsc.subcore_barrier()

    if bidx_h is not None:
        def mrow(i, c):
            w_rows[0][i, pl.ds(0, 16)] = jnp.where(lane == 1, 1.0, 0.0)
            return c
        lax.fori_loop(0, CH, mrow, 0)
        wid = cid * NS + sid
        for j in range(BT // CH):
            pltpu.sync_copy(bidx_h.at[pl.ds(wid * BT + j * CH, CH)], dst_v[0])
            pltpu.sync_copy(w_rows[0], accum_w.at[dst_v[0]], add=True)

    def idx_arrays(nhop):
        return ([dst_h, src_h, ra_h, rb_h] if nhop
                else [dst_n, src_n, rt_n])

    def idx_bufs(p, nhop):
        return ([dstblk[p], srcblk[p], rablk[p], rbblk[p]] if nhop
                else [dstblk[p], srcblk[p], rablk[p]])

    def issue_idx(p, brow, nhop):
        for arr, buf in zip(idx_arrays(nhop), idx_bufs(p, nhop)):
            pltpu.async_copy(arr.at[pl.ds(brow, BLK)], buf, semi[p])

    def drain_idx(p, brow, nhop):
        for arr, buf in zip(idx_arrays(nhop), idx_bufs(p, nhop)):
            pltpu.make_async_copy(arr.at[pl.ds(brow, BLK)], buf,
                                  semi[p]).wait()

    def gather_list(b, p, k, nhop):
        pairs = [
            (psrc_hbm.at[srcblk[p].at[k]], rows_s[b]),
            (prel_hbm.at[rablk[p].at[k]], rows_a[b]),
            (tdst_hbm.at[dstblk[p].at[k]], dscal[b]),
            (tsrc_hbm.at[srcblk[p].at[k]], sscal[b]),
            (trel_hbm.at[rablk[p].at[k]], rscal[b]),
        ]
        if nhop:
            pairs += [(prel_hbm.at[rbblk[p].at[k]], rows_b[b]),
                      (trel_hbm.at[rbblk[p].at[k]], rscalb[b])]
        return pairs

    def issue(b, p, k, nhop):
        for s, d in gather_list(b, p, k, nhop):
            pltpu.async_copy(s, d, sem[b])

    def drain(b, p, k, nhop):
        for s, d in gather_list(b, p, k, nhop):
            pltpu.make_async_copy(s, d, sem[b]).wait()

    def drain_scatter(b):
        pltpu.make_async_copy(out_rows[b], accum_n.at[dst_v[b]],
                              semw[b]).wait()
        pltpu.make_async_copy(w_rows[b], accum_w.at[dst_v[b]],
                              semw[b]).wait()

    def compute(b, p, k, nhop):
        for g in range(CH // 16):
            dst_v[b][pl.ds(g * 16, 16)] = dstblk[p][k, pl.ds(g * 16, 16)]

        def eb(e, c):
            sl = pl.ds(0, 16)
            z16 = dscal[b][e, sl] + sscal[b][e, sl] + rscal[b][e, sl]
            if nhop:
                z16 = z16 + rscalb[b][e, sl]
            w16 = _leakyexp(z16)
            w_rows[b][e, sl] = jnp.where(lane < nheads, w16, 0.0)
            esplat = jnp.full((16,), e, jnp.int32)
            w0 = plsc.load_gather(w_rows[b], [esplat, zeros16])
            if nheads == 2:
                w1 = plsc.load_gather(w_rows[b], [esplat, zeros16 + 1])
            for d in range(8):
                sld = pl.ds(d * 16, 16)
                v = rows_s[b][e, sld] + rows_a[b][e, sld]
                if nhop:
                    v = v + rows_b[b][e, sld]
                if nheads == 2:
                    out_rows[b][e, sld] = v * (w0 if d < 4 else w1)
                else:
                    out_rows[b][e, sld] = v * w0
            return c
        lax.fori_loop(0, CH, eb, 0)
        pltpu.async_copy(out_rows[b], accum_n.at[dst_v[b]], semw[b], add=True)
        pltpu.async_copy(w_rows[b], accum_w.at[dst_v[b]], semw[b], add=True)

    def edge_loop(nchunks, cbase, nhop, first_loop):
        nblocks = nchunks // BLK
        crow = cbase // CH
        issue_idx(0, crow, nhop)
        drain_idx(0, crow, nhop)
        issue(0, 0, 0, nhop)
        issue_idx(1, crow + BLK, nhop)

        def block_pair(i2, c):
            for parity in range(2):
                ib = i2 * 2 + parity
                brow = crow + ib * BLK
                for k in range(BLK):
                    b = k % 2
                    drain(b, parity, k, nhop)
                    if k + 1 < BLK:
                        issue(1 - b, parity, k + 1, nhop)
                    else:
                        @pl.when(ib + 1 < nblocks)
                        def _():
                            drain_idx(1 - parity, brow + BLK, nhop)
                            issue(1 - b, 1 - parity, 0, nhop)
                    if first_loop:
                        @pl.when(ib * BLK + k >= 2)
                        def _():
                            drain_scatter(b)
                    else:
                        drain_scatter(b)
                    compute(b, parity, k, nhop)

                @pl.when(ib + 2 < nblocks)
                def _():
                    issue_idx(parity, brow + 2 * BLK, nhop)
            return c
        lax.fori_loop(0, nblocks // 2, block_pair, 0)

    edge_loop(ET_N // CH, cid * EC_N + sid * ET_N, False, True)
    edge_loop(ET_H // CH, cid * EC_H + sid * ET_H, True, False)
    drain_scatter(0)
    drain_scatter(1)

    plsc.subcore_barrier()
    for j in range(ROWS_PER_TILE // CH):
        st = sid * ROWS_PER_TILE + j * CH
        pltpu.sync_copy(accum_n.at[pl.ds(st, CH)], out_rows[0])
        pltpu.sync_copy(out_rows[0], out_n.at[pl.ds(cid * NPAD + st, CH)])
        pltpu.sync_copy(accum_w.at[pl.ds(st, CH)], w_rows[0])
        pltpu.sync_copy(w_rows[0], out_w.at[pl.ds(cid * NPAD + st, CH)])


def _att1_body(dst_n, src_n, rt_n, dst_h, src_h, ra_h, rb_h,
               psrc_hbm, prel_hbm, tdst_hbm, tsrc_hbm, trel_hbm,
               out_n, out_w, *rest):
    _att_body(2, dst_n, src_n, rt_n, dst_h, src_h, ra_h, rb_h, None,
              psrc_hbm, prel_hbm, tdst_hbm, tsrc_hbm, trel_hbm,
              out_n, out_w, *rest)


def _att2_body(dst_n, src_n, rt_n, dst_h, src_h, ra_h, rb_h, bidx_h,
               psrc_hbm, prel_hbm, tdst_hbm, tsrc_hbm, trel_hbm,
               out_n, out_w, *rest):
    _att_body(1, dst_n, src_n, rt_n, dst_h, src_h, ra_h, rb_h, bidx_h,
              psrc_hbm, prel_hbm, tdst_hbm, tsrc_hbm, trel_hbm,
              out_n, out_w, *rest)


def _mesh():
    return plsc.VectorSubcoreMesh(core_axis_name="c", subcore_axis_name="s")


_OUT_TYPE = (jax.ShapeDtypeStruct((NC * NPAD, D), jnp.float32),
             jax.ShapeDtypeStruct((NC * NPAD, 16), jnp.float32))

_SCRATCH = (
    [pltpu.VMEM_SHARED((NPAD, D), jnp.float32),
     pltpu.VMEM_SHARED((NPAD, 16), jnp.float32)]
    + [pltpu.VMEM((BLK, CH), jnp.int32)] * 8
    + [pltpu.VMEM((CH,), jnp.int32)] * 2
    + [pltpu.VMEM((CH, D), jnp.float32)] * 6
    + [pltpu.VMEM((CH, 16), jnp.float32)] * 8
    + [pltpu.VMEM((CH, D), jnp.float32)] * 2
    + [pltpu.VMEM((CH, 16), jnp.float32)] * 2
    + [pltpu.SemaphoreType.DMA] * 6
)

_att1 = pl.kernel(
    _att1_body, mesh=_mesh(),
    compiler_params=pltpu.CompilerParams(
        needs_layout_passes=False, use_tc_tiling_on_sc=False),
    out_type=_OUT_TYPE, scratch_types=_SCRATCH)

_att2 = pl.kernel(
    _att2_body, mesh=_mesh(),
    compiler_params=pltpu.CompilerParams(
        needs_layout_passes=False, use_tc_tiling_on_sc=False),
    out_type=_OUT_TYPE, scratch_types=_SCRATCH)


def _normalize_rows(x):
    n = jnp.linalg.norm(x, axis=1, keepdims=True)
    return x / jnp.maximum(n, 1e-12)


def _pad_to(x, n, value=0):
    return jnp.pad(x, [(0, n - x.shape[0])] + [(0, 0)] * (x.ndim - 1),
                   constant_values=value)


def _scal16(*cols):
    rows = cols[0].shape[0]
    out = jnp.zeros((rows, 16), jnp.float32)
    for i, c in enumerate(cols):
        out = out.at[:, i].set(c)
    return out


@jax.jit
def kernel(Corpus_, batch_inputs, edge_list, edge_type, train_indices_nhop,
           entity_embeddings, relation_embeddings, W_entities, W_spgat,
           a_heads, a2_heads, a_out, a2_out):
    f32 = jnp.float32
    ent = _normalize_rows(entity_embeddings)
    rel = _normalize_rows(relation_embeddings)
    rel_aug = jnp.concatenate([rel, jnp.zeros((RPAD - RN, 128), f32)], axis=0)

    i32 = jnp.int32
    dst_n = _pad_to(edge_list[0].astype(i32), EPAD, N).reshape(-1, CH)
    src_n = _pad_to(edge_list[1].astype(i32), EPAD, 0).reshape(-1, CH)
    rt_n = _pad_to(edge_type.astype(i32), EPAD, RN).reshape(-1, CH)
    tin = train_indices_nhop.astype(i32)
    dst_h = _pad_to(tin[:, 3], HPAD, N).reshape(-1, CH)
    src_h = _pad_to(tin[:, 0], HPAD, 0).reshape(-1, CH)
    ra_h = _pad_to(tin[:, 1], HPAD, RN).reshape(-1, CH)
    rb_h = _pad_to(tin[:, 2], HPAD, RN).reshape(-1, CH)
    bidx = batch_inputs[:, 2].astype(i32)

    p_dst = jnp.concatenate(
        [ent @ a_heads[0, :, :128].T, ent @ a_heads[1, :, :128].T], axis=1)
    p_src = jnp.concatenate(
        [ent @ a_heads[0, :, 128:256].T, ent @ a_heads[1, :, 128:256].T], axis=1)
    p_rel = jnp.concatenate(
        [rel_aug @ a_heads[0, :, 256:].T, rel_aug @ a_heads[1, :, 256:].T], axis=1)
    sd0 = p_dst[:, :64] @ a2_heads[0, 0]
    sd1 = p_dst[:, 64:] @ a2_heads[1, 0]
    ss0 = p_src[:, :64] @ a2_heads[0, 0]
    ss1 = p_src[:, 64:] @ a2_heads[1, 0]
    sr0 = p_rel[:, :64] @ a2_heads[0, 0]
    sr1 = p_rel[:, 64:] @ a2_heads[1, 0]
    tdst = _pad_to(_scal16(sd0, sd1), NPAD)
    tsrc = _pad_to(_scal16(ss0, ss1), NPAD)
    trel = _scal16(sr0, sr1)

    acc_n, acc_w = _att1(dst_n, src_n, rt_n, dst_h, src_h, ra_h, rb_h,
                         p_src, p_rel, tdst, tsrc, trel)
    acc_n = acc_n[:NPAD] + acc_n[NPAD:]
    acc_w = acc_w[:NPAD] + acc_w[NPAD:]
    r0 = acc_w[:N, 0:1]
    r1 = acc_w[:N, 1:2]
    h0 = (p_dst[:, :64] * r0 + acc_n[:N, :64]) / jnp.where(r0 == 0.0, 1e-12, r0)
    h1 = (p_dst[:, 64:] * r1 + acc_n[:N, 64:]) / jnp.where(r1 == 0.0, 1e-12, r1)
    x = jnp.concatenate([jax.nn.elu(h0), jax.nn.elu(h1)], axis=1)

    out_relation_1 = rel @ W_spgat
    orel_aug = jnp.concatenate(
        [out_relation_1, jnp.zeros((RPAD - RN, 128), f32)], axis=0)
    q_dst = x @ a_out[:, :128].T
    q_src = x @ a_out[:, 128:256].T
    q_rel = orel_aug @ a_out[:, 256:].T
    s2d = q_dst @ a2_out[0]
    s2s = q_src @ a2_out[0]
    s2r = q_rel @ a2_out[0]
    tdst2 = _pad_to(_scal16(s2d), NPAD)
    tsrc2 = _pad_to(_scal16(s2s), NPAD)
    trel2 = _scal16(s2r)

    acc2_n, acc2_w = _att2(dst_n, src_n, rt_n, dst_h, src_h, ra_h, rb_h, bidx,
                           q_src, q_rel, tdst2, tsrc2, trel2)
    acc2_n = acc2_n[:NPAD] + acc2_n[NPAD:]
    acc2_w = acc2_w[:NPAD] + acc2_w[NPAD:]
    r2 = acc2_w[:N, 0:1]
    h2 = (q_dst * r2 + acc2_n[:N]) / jnp.where(r2 == 0.0, 1e-12, r2)
    x2 = jax.nn.elu(h2)
    mask = (acc2_w[:N, 1:2] > 0.0).astype(f32)

    out_entity_1 = _normalize_rows(ent @ W_entities + mask * x2)
    return (out_entity_1, out_relation_1)


# speedup vs baseline: 3.7854x; 1.0834x over previous
import jax
import jax.numpy as jnp
from jax import lax
from jax.experimental import pallas as pl
from jax.experimental.pallas import tpu as pltpu
from jax.experimental.pallas import tpu_sc as plsc

N = 10000
E = 160000
NHOP = 40000
RN = 500
ALPHA = 0.2

NC = 2
NS = 16
NW = NC * NS

NPAD = 10240
RPAD = 512
EPAD = 163840
HPAD = 40960
CH = 32
D = 128
ROWS_PER_TILE = NPAD // NS
ET_N, EC_N = EPAD // NW, EPAD // NC
ET_H, EC_H = HPAD // NW, HPAD // NC
BT = 4096 // NW


def _leakyexp(z):
    return jnp.exp(-jnp.where(z >= 0, z, ALPHA * z))


def _splat(vec, idx16):
    dnums = lax.GatherDimensionNumbers(
        offset_dims=(), collapsed_slice_dims=(0,), start_index_map=(0,))
    return lax.gather(vec, idx16[:, None], dnums, (1,),
                      mode=lax.GatherScatterMode.PROMISE_IN_BOUNDS)


def _zero_rows(buf, width, nrows):
    def body(i, c):
        for d in range(width // 16):
            buf[i, pl.ds(d * 16, 16)] = jnp.zeros((16,), jnp.float32)
        return c
    lax.fori_loop(0, nrows, body, 0)


BLK = 2


def _att_body(nheads,
              dst_n, src_n, rt_n, dst_h, src_h, ra_h, rb_h, bidx_h,
              psrc_hbm, prel_hbm, tdst_hbm, tsrc_hbm, trel_hbm,
              out_n, out_w,
              accum_n, accum_w,
              dstblk0, dstblk1, srcblk0, srcblk1, rablk0, rablk1,
              rbblk0, rbblk1, dst_v0, dst_v1,
              rows_s0, rows_s1, rows_a0, rows_a1, rows_b0, rows_b1,
              dscal0, dscal1, sscal0, sscal1, rscal0, rscal1,
              rscalb0, rscalb1,
              out_rows0, out_rows1, w_rows0, w_rows1,
              sem0, sem1, semi0, semi1, semw0, semw1):
    cid = lax.axis_index("c")
    sid = lax.axis_index("s")
    lane = lax.iota(jnp.int32, 16)
    zeros16 = jnp.zeros((16,), jnp.int32)
    dstblk = [dstblk0, dstblk1]
    srcblk = [srcblk0, srcblk1]
    rablk = [rablk0, rablk1]
    rbblk = [rbblk0, rbblk1]
    dst_v = [dst_v0, dst_v1]
    rows_s = [rows_s0, rows_s1]
    rows_a = [rows_a0, rows_a1]
    rows_b = [rows_b0, rows_b1]
    dscal = [dscal0, dscal1]
    sscal = [sscal0, sscal1]
    rscal = [rscal0, rscal1]
    rscalb = [rscalb0, rscalb1]
    out_rows = [out_rows0, out_rows1]
    w_rows = [w_rows0, w_rows1]
    sem = [sem0, sem1]
    semi = [semi0, semi1]
    semw = [semw0, semw1]
    SPAN = BLK * CH

    _zero_rows(out_rows[0], D, CH)
    _zero_rows(w_rows[0], 16, CH)
    for j in range(ROWS_PER_TILE // CH):
        st = sid * ROWS_PER_TILE + j * CH
        pltpu.sync_copy(out_rows[0], accum_n.at[pl.ds(st, CH)])
        pltpu.sync_copy(w_rows[0], accum_w.at[pl.ds(st, CH)])
    plsc.subcore_barrier()

    if bidx_h is not None:
        def mrow(i, c):
            w_rows[0][i, pl.ds(0, 16)] = jnp.where(lane == 1, 1.0, 0.0)
            return c
        lax.fori_loop(0, CH, mrow, 0)
        wid = cid * NS + sid
        for j in range(BT // CH):
            pltpu.sync_copy(bidx_h.at[pl.ds(wid * BT + j * CH, CH)], dst_v[0])
            pltpu.sync_copy(w_rows[0], accum_w.at[dst_v[0]], add=True)

    def idx_arrays(nhop):
        return ([dst_h, src_h, ra_h, rb_h] if nhop
                else [dst_n, src_n, rt_n])

    def idx_bufs(p, nhop):
        return ([dstblk[p], srcblk[p], rablk[p], rbblk[p]] if nhop
                else [dstblk[p], srcblk[p], rablk[p]])

    def issue_idx(p, brow, nhop):
        for arr, buf in zip(idx_arrays(nhop), idx_bufs(p, nhop)):
            pltpu.async_copy(arr.at[pl.ds(brow, BLK)], buf, semi[p])

    def drain_idx(p, brow, nhop):
        for arr, buf in zip(idx_arrays(nhop), idx_bufs(p, nhop)):
            pltpu.make_async_copy(arr.at[pl.ds(brow, BLK)], buf,
                                  semi[p]).wait()

    def gather_list(b, p, k, nhop):
        pairs = [
            (psrc_hbm.at[srcblk[p].at[k]], rows_s[b]),
            (prel_hbm.at[rablk[p].at[k]], rows_a[b]),
            (tdst_hbm.at[dstblk[p].at[k]], dscal[b]),
            (tsrc_hbm.at[srcblk[p].at[k]], sscal[b]),
            (trel_hbm.at[rablk[p].at[k]], rscal[b]),
        ]
        if nhop:
            pairs += [(prel_hbm.at[rbblk[p].at[k]], rows_b[b]),
                      (trel_hbm.at[rbblk[p].at[k]], rscalb[b])]
        return pairs

    def issue(b, p, k, nhop):
        for s, d in gather_list(b, p, k, nhop):
            pltpu.async_copy(s, d, sem[b])

    def drain(b, p, k, nhop):
        for s, d in gather_list(b, p, k, nhop):
            pltpu.make_async_copy(s, d, sem[b]).wait()

    def drain_scatter(b):
        pltpu.make_async_copy(out_rows[b], accum_n.at[dst_v[b]],
                              semw[b]).wait()
        pltpu.make_async_copy(w_rows[b], accum_w.at[dst_v[b]],
                              semw[b]).wait()

    def compute(b, p, k, nhop):
        for g in range(CH // 16):
            dst_v[b][pl.ds(g * 16, 16)] = dstblk[p][k, pl.ds(g * 16, 16)]

        def eb(e, c):
            sl = pl.ds(0, 16)
            z16 = dscal[b][e, sl] + sscal[b][e, sl] + rscal[b][e, sl]
            if nhop:
                z16 = z16 + rscalb[b][e, sl]
            w16 = _leakyexp(z16)
            w_rows[b][e, sl] = jnp.where(lane < nheads, w16, 0.0)
            w0 = _splat(w16, zeros16)
            if nheads == 2:
                w1 = _splat(w16, zeros16 + 1)
            for d in range(8):
                sld = pl.ds(d * 16, 16)
                v = rows_s[b][e, sld] + rows_a[b][e, sld]
                if nhop:
                    v = v + rows_b[b][e, sld]
                if nheads == 2:
                    out_rows[b][e, sld] = v * (w0 if d < 4 else w1)
                else:
                    out_rows[b][e, sld] = v * w0
        plsc.parallel_loop(0, CH, unroll=4)(lambda e: eb(e, 0))
        pltpu.async_copy(out_rows[b], accum_n.at[dst_v[b]], semw[b], add=True)
        pltpu.async_copy(w_rows[b], accum_w.at[dst_v[b]], semw[b], add=True)

    def edge_loop(nchunks, cbase, nhop, first_loop):
        nblocks = nchunks // BLK
        crow = cbase // CH
        issue_idx(0, crow, nhop)
        drain_idx(0, crow, nhop)
        issue(0, 0, 0, nhop)
        issue_idx(1, crow + BLK, nhop)

        def block_pair(i2, c):
            for parity in range(2):
                ib = i2 * 2 + parity
                brow = crow + ib * BLK
                for k in range(BLK):
                    b = k % 2
                    drain(b, parity, k, nhop)
                    if k + 1 < BLK:
                        issue(1 - b, parity, k + 1, nhop)
                    else:
                        @pl.when(ib + 1 < nblocks)
                        def _():
                            drain_idx(1 - parity, brow + BLK, nhop)
                            issue(1 - b, 1 - parity, 0, nhop)
                    if first_loop:
                        @pl.when(ib * BLK + k >= 2)
                        def _():
                            drain_scatter(b)
                    else:
                        drain_scatter(b)
                    compute(b, parity, k, nhop)

                @pl.when(ib + 2 < nblocks)
                def _():
                    issue_idx(parity, brow + 2 * BLK, nhop)
            return c
        lax.fori_loop(0, nblocks // 2, block_pair, 0)

    edge_loop(ET_N // CH, cid * EC_N + sid * ET_N, False, True)
    edge_loop(ET_H // CH, cid * EC_H + sid * ET_H, True, False)
    drain_scatter(0)
    drain_scatter(1)

    plsc.subcore_barrier()
    for j in range(ROWS_PER_TILE // CH):
        st = sid * ROWS_PER_TILE + j * CH
        pltpu.sync_copy(accum_n.at[pl.ds(st, CH)], out_rows[0])
        pltpu.sync_copy(out_rows[0], out_n.at[pl.ds(cid * NPAD + st, CH)])
        pltpu.sync_copy(accum_w.at[pl.ds(st, CH)], w_rows[0])
        pltpu.sync_copy(w_rows[0], out_w.at[pl.ds(cid * NPAD + st, CH)])


def _att1_body(dst_n, src_n, rt_n, dst_h, src_h, ra_h, rb_h,
               psrc_hbm, prel_hbm, tdst_hbm, tsrc_hbm, trel_hbm,
               out_n, out_w, *rest):
    _att_body(2, dst_n, src_n, rt_n, dst_h, src_h, ra_h, rb_h, None,
              psrc_hbm, prel_hbm, tdst_hbm, tsrc_hbm, trel_hbm,
              out_n, out_w, *rest)


def _att2_body(dst_n, src_n, rt_n, dst_h, src_h, ra_h, rb_h, bidx_h,
               psrc_hbm, prel_hbm, tdst_hbm, tsrc_hbm, trel_hbm,
               out_n, out_w, *rest):
    _att_body(1, dst_n, src_n, rt_n, dst_h, src_h, ra_h, rb_h, bidx_h,
              psrc_hbm, prel_hbm, tdst_hbm, tsrc_hbm, trel_hbm,
              out_n, out_w, *rest)


def _mesh():
    return plsc.VectorSubcoreMesh(core_axis_name="c", subcore_axis_name="s")


_OUT_TYPE = (jax.ShapeDtypeStruct((NC * NPAD, D), jnp.float32),
             jax.ShapeDtypeStruct((NC * NPAD, 16), jnp.float32))

_SCRATCH = (
    [pltpu.VMEM_SHARED((NPAD, D), jnp.float32),
     pltpu.VMEM_SHARED((NPAD, 16), jnp.float32)]
    + [pltpu.VMEM((BLK, CH), jnp.int32)] * 8
    + [pltpu.VMEM((CH,), jnp.int32)] * 2
    + [pltpu.VMEM((CH, D), jnp.float32)] * 6
    + [pltpu.VMEM((CH, 16), jnp.float32)] * 8
    + [pltpu.VMEM((CH, D), jnp.float32)] * 2
    + [pltpu.VMEM((CH, 16), jnp.float32)] * 2
    + [pltpu.SemaphoreType.DMA] * 6
)

_att1 = pl.kernel(
    _att1_body, mesh=_mesh(),
    compiler_params=pltpu.CompilerParams(
        needs_layout_passes=False, use_tc_tiling_on_sc=False),
    out_type=_OUT_TYPE, scratch_types=_SCRATCH)

_att2 = pl.kernel(
    _att2_body, mesh=_mesh(),
    compiler_params=pltpu.CompilerParams(
        needs_layout_passes=False, use_tc_tiling_on_sc=False),
    out_type=_OUT_TYPE, scratch_types=_SCRATCH)


def _normalize_rows(x):
    n = jnp.linalg.norm(x, axis=1, keepdims=True)
    return x / jnp.maximum(n, 1e-12)


def _pad_to(x, n, value=0):
    return jnp.pad(x, [(0, n - x.shape[0])] + [(0, 0)] * (x.ndim - 1),
                   constant_values=value)


def _scal16(*cols):
    rows = cols[0].shape[0]
    out = jnp.zeros((rows, 16), jnp.float32)
    for i, c in enumerate(cols):
        out = out.at[:, i].set(c)
    return out


@jax.jit
def kernel(Corpus_, batch_inputs, edge_list, edge_type, train_indices_nhop,
           entity_embeddings, relation_embeddings, W_entities, W_spgat,
           a_heads, a2_heads, a_out, a2_out):
    f32 = jnp.float32
    ent = _normalize_rows(entity_embeddings)
    rel = _normalize_rows(relation_embeddings)
    rel_aug = jnp.concatenate([rel, jnp.zeros((RPAD - RN, 128), f32)], axis=0)

    i32 = jnp.int32
    dst_n = _pad_to(edge_list[0].astype(i32), EPAD, N).reshape(-1, CH)
    src_n = _pad_to(edge_list[1].astype(i32), EPAD, 0).reshape(-1, CH)
    rt_n = _pad_to(edge_type.astype(i32), EPAD, RN).reshape(-1, CH)
    tin = train_indices_nhop.astype(i32)
    dst_h = _pad_to(tin[:, 3], HPAD, N).reshape(-1, CH)
    src_h = _pad_to(tin[:, 0], HPAD, 0).reshape(-1, CH)
    ra_h = _pad_to(tin[:, 1], HPAD, RN).reshape(-1, CH)
    rb_h = _pad_to(tin[:, 2], HPAD, RN).reshape(-1, CH)
    bidx = batch_inputs[:, 2].astype(i32)

    p_dst = jnp.concatenate(
        [ent @ a_heads[0, :, :128].T, ent @ a_heads[1, :, :128].T], axis=1)
    p_src = jnp.concatenate(
        [ent @ a_heads[0, :, 128:256].T, ent @ a_heads[1, :, 128:256].T], axis=1)
    p_rel = jnp.concatenate(
        [rel_aug @ a_heads[0, :, 256:].T, rel_aug @ a_heads[1, :, 256:].T], axis=1)
    sd0 = p_dst[:, :64] @ a2_heads[0, 0]
    sd1 = p_dst[:, 64:] @ a2_heads[1, 0]
    ss0 = p_src[:, :64] @ a2_heads[0, 0]
    ss1 = p_src[:, 64:] @ a2_heads[1, 0]
    sr0 = p_rel[:, :64] @ a2_heads[0, 0]
    sr1 = p_rel[:, 64:] @ a2_heads[1, 0]
    tdst = _pad_to(_scal16(sd0, sd1), NPAD)
    tsrc = _pad_to(_scal16(ss0, ss1), NPAD)
    trel = _scal16(sr0, sr1)

    acc_n, acc_w = _att1(dst_n, src_n, rt_n, dst_h, src_h, ra_h, rb_h,
                         p_src, p_rel, tdst, tsrc, trel)
    acc_n = acc_n[:NPAD] + acc_n[NPAD:]
    acc_w = acc_w[:NPAD] + acc_w[NPAD:]
    r0 = acc_w[:N, 0:1]
    r1 = acc_w[:N, 1:2]
    h0 = (p_dst[:, :64] * r0 + acc_n[:N, :64]) / jnp.where(r0 == 0.0, 1e-12, r0)
    h1 = (p_dst[:, 64:] * r1 + acc_n[:N, 64:]) / jnp.where(r1 == 0.0, 1e-12, r1)
    x = jnp.concatenate([jax.nn.elu(h0), jax.nn.elu(h1)], axis=1)

    out_relation_1 = rel @ W_spgat
    orel_aug = jnp.concatenate(
        [out_relation_1, jnp.zeros((RPAD - RN, 128), f32)], axis=0)
    q_dst = x @ a_out[:, :128].T
    q_src = x @ a_out[:, 128:256].T
    q_rel = orel_aug @ a_out[:, 256:].T
    s2d = q_dst @ a2_out[0]
    s2s = q_src @ a2_out[0]
    s2r = q_rel @ a2_out[0]
    tdst2 = _pad_to(_scal16(s2d), NPAD)
    tsrc2 = _pad_to(_scal16(s2s), NPAD)
    trel2 = _scal16(s2r)

    acc2_n, acc2_w = _att2(dst_n, src_n, rt_n, dst_h, src_h, ra_h, rb_h, bidx,
                           q_src, q_rel, tdst2, tsrc2, trel2)
    acc2_n = acc2_n[:NPAD] + acc2_n[NPAD:]
    acc2_w = acc2_w[:NPAD] + acc2_w[NPAD:]
    r2 = acc2_w[:N, 0:1]
    h2 = (q_dst * r2 + acc2_n[:N]) / jnp.where(r2 == 0.0, 1e-12, r2)
    x2 = jax.nn.elu(h2)
    mask = (acc2_w[:N, 1:2] > 0.0).astype(f32)

    out_entity_1 = _normalize_rows(ent @ W_entities + mask * x2)
    return (out_entity_1, out_relation_1)


# speedup vs baseline: 3.8750x; 1.0236x over previous
import jax
import jax.numpy as jnp
from jax import lax
from jax.experimental import pallas as pl
from jax.experimental.pallas import tpu as pltpu
from jax.experimental.pallas import tpu_sc as plsc

N = 10000
E = 160000
NHOP = 40000
RN = 500
ALPHA = 0.2

NC = 2
NS = 16
NW = NC * NS

NPAD = 10240
RPAD = 512
EPAD = 163840
HPAD = 40960
CH = 32
D = 128
DG = 144
ROWS_PER_TILE = NPAD // NS
ET_N, EC_N = EPAD // NW, EPAD // NC
ET_H, EC_H = HPAD // NW, HPAD // NC
BT = 4096 // NW


def _leakyexp(z):
    return jnp.exp(-jnp.where(z >= 0, z, ALPHA * z))


def _splat(vec, idx16):
    dnums = lax.GatherDimensionNumbers(
        offset_dims=(), collapsed_slice_dims=(0,), start_index_map=(0,))
    return lax.gather(vec, idx16[:, None], dnums, (1,),
                      mode=lax.GatherScatterMode.PROMISE_IN_BOUNDS)


def _zero_rows(buf, width, nrows):
    def body(i, c):
        for d in range(width // 16):
            buf[i, pl.ds(d * 16, 16)] = jnp.zeros((16,), jnp.float32)
        return c
    lax.fori_loop(0, nrows, body, 0)


BLK = 2


def _att_body(nheads,
              dst_n, src_n, rt_n, dst_h, src_h, ra_h, rb_h, bidx_h,
              psrc_hbm, prel_hbm, tdst_hbm,
              out_n, out_w,
              accum_n, accum_w,
              dstblk0, dstblk1, srcblk0, srcblk1, rablk0, rablk1,
              rbblk0, rbblk1, dst_v0, dst_v1,
              rows_s0, rows_s1, rows_a0, rows_a1, rows_b0, rows_b1,
              dscal0, dscal1,
              out_rows0, out_rows1, w_rows0, w_rows1,
              sem0, sem1, semi0, semi1, semw0, semw1):
    cid = lax.axis_index("c")
    sid = lax.axis_index("s")
    lane = lax.iota(jnp.int32, 16)
    zeros16 = jnp.zeros((16,), jnp.int32)
    dstblk = [dstblk0, dstblk1]
    srcblk = [srcblk0, srcblk1]
    rablk = [rablk0, rablk1]
    rbblk = [rbblk0, rbblk1]
    dst_v = [dst_v0, dst_v1]
    rows_s = [rows_s0, rows_s1]
    rows_a = [rows_a0, rows_a1]
    rows_b = [rows_b0, rows_b1]
    dscal = [dscal0, dscal1]
    out_rows = [out_rows0, out_rows1]
    w_rows = [w_rows0, w_rows1]
    sem = [sem0, sem1]
    semi = [semi0, semi1]
    semw = [semw0, semw1]
    SPAN = BLK * CH

    _zero_rows(out_rows[0], D, CH)
    _zero_rows(w_rows[0], 16, CH)
    for j in range(ROWS_PER_TILE // CH):
        st = sid * ROWS_PER_TILE + j * CH
        pltpu.sync_copy(out_rows[0], accum_n.at[pl.ds(st, CH)])
        pltpu.sync_copy(w_rows[0], accum_w.at[pl.ds(st, CH)])
    plsc.subcore_barrier()

    if bidx_h is not None:
        def mrow(i, c):
            w_rows[0][i, pl.ds(0, 16)] = jnp.where(lane == 1, 1.0, 0.0)
            return c
        lax.fori_loop(0, CH, mrow, 0)
        wid = cid * NS + sid
        for j in range(BT // CH):
            pltpu.sync_copy(bidx_h.at[pl.ds(wid * BT + j * CH, CH)], dst_v[0])
            pltpu.sync_copy(w_rows[0], accum_w.at[dst_v[0]], add=True)

    def idx_arrays(nhop):
        return ([dst_h, src_h, ra_h, rb_h] if nhop
                else [dst_n, src_n, rt_n])

    def idx_bufs(p, nhop):
        return ([dstblk[p], srcblk[p], rablk[p], rbblk[p]] if nhop
                else [dstblk[p], srcblk[p], rablk[p]])

    def issue_idx(p, brow, nhop):
        for arr, buf in zip(idx_arrays(nhop), idx_bufs(p, nhop)):
            pltpu.async_copy(arr.at[pl.ds(brow, BLK)], buf, semi[p])

    def drain_idx(p, brow, nhop):
        for arr, buf in zip(idx_arrays(nhop), idx_bufs(p, nhop)):
            pltpu.make_async_copy(arr.at[pl.ds(brow, BLK)], buf,
                                  semi[p]).wait()

    def gather_list(b, p, k, nhop):
        pairs = [
            (psrc_hbm.at[srcblk[p].at[k]], rows_s[b]),
            (prel_hbm.at[rablk[p].at[k]], rows_a[b]),
            (tdst_hbm.at[dstblk[p].at[k]], dscal[b]),
        ]
        if nhop:
            pairs += [(prel_hbm.at[rbblk[p].at[k]], rows_b[b])]
        return pairs

    def issue(b, p, k, nhop):
        for s, d in gather_list(b, p, k, nhop):
            pltpu.async_copy(s, d, sem[b])

    def drain(b, p, k, nhop):
        for s, d in gather_list(b, p, k, nhop):
            pltpu.make_async_copy(s, d, sem[b]).wait()

    def drain_scatter(b):
        pltpu.make_async_copy(out_rows[b], accum_n.at[dst_v[b]],
                              semw[b]).wait()
        pltpu.make_async_copy(w_rows[b], accum_w.at[dst_v[b]],
                              semw[b]).wait()

    def compute(b, p, k, nhop):
        for g in range(CH // 16):
            dst_v[b][pl.ds(g * 16, 16)] = dstblk[p][k, pl.ds(g * 16, 16)]

        def eb(e, c):
            sl = pl.ds(0, 16)
            slz = pl.ds(D, 16)
            z16 = dscal[b][e, sl] + rows_s[b][e, slz] + rows_a[b][e, slz]
            if nhop:
                z16 = z16 + rows_b[b][e, slz]
            w16 = _leakyexp(z16)
            w_rows[b][e, sl] = jnp.where(lane < nheads, w16, 0.0)
            w0 = _splat(w16, zeros16)
            if nheads == 2:
                w1 = _splat(w16, zeros16 + 1)
            for d in range(8):
                sld = pl.ds(d * 16, 16)
                v = rows_s[b][e, sld] + rows_a[b][e, sld]
                if nhop:
                    v = v + rows_b[b][e, sld]
                if nheads == 2:
                    out_rows[b][e, sld] = v * (w0 if d < 4 else w1)
                else:
                    out_rows[b][e, sld] = v * w0
        plsc.parallel_loop(0, CH, unroll=4)(lambda e: eb(e, 0))
        pltpu.async_copy(out_rows[b], accum_n.at[dst_v[b]], semw[b], add=True)
        pltpu.async_copy(w_rows[b], accum_w.at[dst_v[b]], semw[b], add=True)

    def edge_loop(nchunks, cbase, nhop, first_loop):
        nblocks = nchunks // BLK
        crow = cbase // CH
        issue_idx(0, crow, nhop)
        drain_idx(0, crow, nhop)
        issue(0, 0, 0, nhop)
        issue_idx(1, crow + BLK, nhop)

        def block_pair(i2, c):
            for parity in range(2):
                ib = i2 * 2 + parity
                brow = crow + ib * BLK
                for k in range(BLK):
                    b = k % 2
                    drain(b, parity, k, nhop)
                    if k + 1 < BLK:
                        issue(1 - b, parity, k + 1, nhop)
                    else:
                        @pl.when(ib + 1 < nblocks)
                        def _():
                            drain_idx(1 - parity, brow + BLK, nhop)
                            issue(1 - b, 1 - parity, 0, nhop)
                    if first_loop:
                        @pl.when(ib * BLK + k >= 2)
                        def _():
                            drain_scatter(b)
                    else:
                        drain_scatter(b)
                    compute(b, parity, k, nhop)

                @pl.when(ib + 2 < nblocks)
                def _():
                    issue_idx(parity, brow + 2 * BLK, nhop)
            return c
        lax.fori_loop(0, nblocks // 2, block_pair, 0)

    edge_loop(ET_N // CH, cid * EC_N + sid * ET_N, False, True)
    edge_loop(ET_H // CH, cid * EC_H + sid * ET_H, True, False)
    drain_scatter(0)
    drain_scatter(1)

    plsc.subcore_barrier()
    for j in range(ROWS_PER_TILE // CH):
        st = sid * ROWS_PER_TILE + j * CH
        pltpu.sync_copy(accum_n.at[pl.ds(st, CH)], out_rows[0])
        pltpu.sync_copy(out_rows[0], out_n.at[pl.ds(cid * NPAD + st, CH)])
        pltpu.sync_copy(accum_w.at[pl.ds(st, CH)], w_rows[0])
        pltpu.sync_copy(w_rows[0], out_w.at[pl.ds(cid * NPAD + st, CH)])


def _att1_body(dst_n, src_n, rt_n, dst_h, src_h, ra_h, rb_h,
               psrc_hbm, prel_hbm, tdst_hbm,
               out_n, out_w, *rest):
    _att_body(2, dst_n, src_n, rt_n, dst_h, src_h, ra_h, rb_h, None,
              psrc_hbm, prel_hbm, tdst_hbm,
              out_n, out_w, *rest)


def _att2_body(dst_n, src_n, rt_n, dst_h, src_h, ra_h, rb_h, bidx_h,
               psrc_hbm, prel_hbm, tdst_hbm,
               out_n, out_w, *rest):
    _att_body(1, dst_n, src_n, rt_n, dst_h, src_h, ra_h, rb_h, bidx_h,
              psrc_hbm, prel_hbm, tdst_hbm,
              out_n, out_w, *rest)


def _mesh():
    return plsc.VectorSubcoreMesh(core_axis_name="c", subcore_axis_name="s")


_OUT_TYPE = (jax.ShapeDtypeStruct((NC * NPAD, D), jnp.float32),
             jax.ShapeDtypeStruct((NC * NPAD, 16), jnp.float32))

_SCRATCH = (
    [pltpu.VMEM_SHARED((NPAD, D), jnp.float32),
     pltpu.VMEM_SHARED((NPAD, 16), jnp.float32)]
    + [pltpu.VMEM((BLK, CH), jnp.int32)] * 8
    + [pltpu.VMEM((CH,), jnp.int32)] * 2
    + [pltpu.VMEM((CH, DG), jnp.float32)] * 6
    + [pltpu.VMEM((CH, 16), jnp.float32)] * 2
    + [pltpu.VMEM((CH, D), jnp.float32)] * 2
    + [pltpu.VMEM((CH, 16), jnp.float32)] * 2
    + [pltpu.SemaphoreType.DMA] * 6
)

_att1 = pl.kernel(
    _att1_body, mesh=_mesh(),
    compiler_params=pltpu.CompilerParams(
        needs_layout_passes=False, use_tc_tiling_on_sc=False),
    out_type=_OUT_TYPE, scratch_types=_SCRATCH)

_att2 = pl.kernel(
    _att2_body, mesh=_mesh(),
    compiler_params=pltpu.CompilerParams(
        needs_layout_passes=False, use_tc_tiling_on_sc=False),
    out_type=_OUT_TYPE, scratch_types=_SCRATCH)


def _normalize_rows(x):
    n = jnp.linalg.norm(x, axis=1, keepdims=True)
    return x / jnp.maximum(n, 1e-12)


def _pad_to(x, n, value=0):
    return jnp.pad(x, [(0, n - x.shape[0])] + [(0, 0)] * (x.ndim - 1),
                   constant_values=value)


def _scal16(*cols):
    rows = cols[0].shape[0]
    out = jnp.zeros((rows, 16), jnp.float32)
    for i, c in enumerate(cols):
        out = out.at[:, i].set(c)
    return out


@jax.jit
def kernel(Corpus_, batch_inputs, edge_list, edge_type, train_indices_nhop,
           entity_embeddings, relation_embeddings, W_entities, W_spgat,
           a_heads, a2_heads, a_out, a2_out):
    f32 = jnp.float32
    ent = _normalize_rows(entity_embeddings)
    rel = _normalize_rows(relation_embeddings)
    rel_aug = jnp.concatenate([rel, jnp.zeros((RPAD - RN, 128), f32)], axis=0)

    i32 = jnp.int32
    dst_n = _pad_to(edge_list[0].astype(i32), EPAD, N).reshape(-1, CH)
    src_n = _pad_to(edge_list[1].astype(i32), EPAD, 0).reshape(-1, CH)
    rt_n = _pad_to(edge_type.astype(i32), EPAD, RN).reshape(-1, CH)
    tin = train_indices_nhop.astype(i32)
    dst_h = _pad_to(tin[:, 3], HPAD, N).reshape(-1, CH)
    src_h = _pad_to(tin[:, 0], HPAD, 0).reshape(-1, CH)
    ra_h = _pad_to(tin[:, 1], HPAD, RN).reshape(-1, CH)
    rb_h = _pad_to(tin[:, 2], HPAD, RN).reshape(-1, CH)
    bidx = batch_inputs[:, 2].astype(i32)

    p_dst = jnp.concatenate(
        [ent @ a_heads[0, :, :128].T, ent @ a_heads[1, :, :128].T], axis=1)
    p_src = jnp.concatenate(
        [ent @ a_heads[0, :, 128:256].T, ent @ a_heads[1, :, 128:256].T], axis=1)
    p_rel = jnp.concatenate(
        [rel_aug @ a_heads[0, :, 256:].T, rel_aug @ a_heads[1, :, 256:].T], axis=1)
    sd0 = p_dst[:, :64] @ a2_heads[0, 0]
    sd1 = p_dst[:, 64:] @ a2_heads[1, 0]
    ss0 = p_src[:, :64] @ a2_heads[0, 0]
    ss1 = p_src[:, 64:] @ a2_heads[1, 0]
    sr0 = p_rel[:, :64] @ a2_heads[0, 0]
    sr1 = p_rel[:, 64:] @ a2_heads[1, 0]
    tdst = _pad_to(_scal16(sd0, sd1), NPAD)
    psrc_t = jnp.concatenate([p_src, _scal16(ss0, ss1)], axis=1)
    prel_t = jnp.concatenate([p_rel, _scal16(sr0, sr1)], axis=1)

    acc_n, acc_w = _att1(dst_n, src_n, rt_n, dst_h, src_h, ra_h, rb_h,
                         psrc_t, prel_t, tdst)
    acc_n = acc_n[:NPAD] + acc_n[NPAD:]
    acc_w = acc_w[:NPAD] + acc_w[NPAD:]
    r0 = acc_w[:N, 0:1]
    r1 = acc_w[:N, 1:2]
    h0 = (p_dst[:, :64] * r0 + acc_n[:N, :64]) / jnp.where(r0 == 0.0, 1e-12, r0)
    h1 = (p_dst[:, 64:] * r1 + acc_n[:N, 64:]) / jnp.where(r1 == 0.0, 1e-12, r1)
    x = jnp.concatenate([jax.nn.elu(h0), jax.nn.elu(h1)], axis=1)

    out_relation_1 = rel @ W_spgat
    orel_aug = jnp.concatenate(
        [out_relation_1, jnp.zeros((RPAD - RN, 128), f32)], axis=0)
    q_dst = x @ a_out[:, :128].T
    q_src = x @ a_out[:, 128:256].T
    q_rel = orel_aug @ a_out[:, 256:].T
    s2d = q_dst @ a2_out[0]
    s2s = q_src @ a2_out[0]
    s2r = q_rel @ a2_out[0]
    tdst2 = _pad_to(_scal16(s2d), NPAD)
    qsrc_t = jnp.concatenate([q_src, _scal16(s2s)], axis=1)
    qrel_t = jnp.concatenate([q_rel, _scal16(s2r)], axis=1)

    acc2_n, acc2_w = _att2(dst_n, src_n, rt_n, dst_h, src_h, ra_h, rb_h, bidx,
                           qsrc_t, qrel_t, tdst2)
    acc2_n = acc2_n[:NPAD] + acc2_n[NPAD:]
    acc2_w = acc2_w[:NPAD] + acc2_w[NPAD:]
    r2 = acc2_w[:N, 0:1]
    h2 = (q_dst * r2 + acc2_n[:N]) / jnp.where(r2 == 0.0, 1e-12, r2)
    x2 = jax.nn.elu(h2)
    mask = (acc2_w[:N, 1:2] > 0.0).astype(f32)

    out_entity_1 = _normalize_rows(ent @ W_entities + mask * x2)
    return (out_entity_1, out_relation_1)
